# baseline (speedup 1.0000x reference)
import functools

import jax
import jax.numpy as jnp
from jax import lax
from jax.experimental import pallas as pl
from jax.experimental.pallas import tpu as pltpu
from jax.experimental.pallas import tpu_sc as plsc

N = 10000
E = 160000
D_IN = 256
HID = 512
OUT = 40
DEC = 2

NTILES = 16
EPT = E // NTILES
NV = EPT // 16
STR = 624
B_E = 100
NQ = 4
EPQ = EPT // NQ
NBQ = EPQ // B_E
NBUF = 3

BN = 1000
NBLK = N // BN

_SC_MESH = dict(core_axis_name="c", subcore_axis_name="s",
                num_cores=2, num_subcores=NTILES)


def _dense1_body(x_ref, w_ref, b_ref, a_ref, dw_ref, db_ref,
                 h_ref, psd_ref, dec_ref):
    c = pl.program_id(1)
    nch = pl.num_programs(1)
    xb = x_ref[...]
    hc = jnp.dot(xb, w_ref[...], preferred_element_type=jnp.float32) + b_ref[...]
    h_ref[0, :, :] = hc
    pc = jnp.dot(hc, a_ref[...], preferred_element_type=jnp.float32)

    @pl.when(c == 0)
    def _():
        psd_ref[...] = pc
        dec_ref[...] = (jnp.dot(xb, dw_ref[...], preferred_element_type=jnp.float32)
                        + db_ref[...])

    @pl.when(c > 0)
    def _():
        psd_ref[...] += pc

    @pl.when(c == nch - 1)
    def _():
        z = dec_ref[...]
        m = jnp.max(z, axis=-1, keepdims=True)
        ez = jnp.exp(z - m)
        dec_ref[...] = ez / jnp.sum(ez, axis=-1, keepdims=True)


def _dense1(x, W1, b1, A1, dW1, db1):
    ch = HID // 128
    return pl.pallas_call(
        _dense1_body,
        grid=(NBLK, ch),
        in_specs=[
            pl.BlockSpec((BN, D_IN), lambda i, c: (i, 0)),
            pl.BlockSpec((D_IN, 128), lambda i, c: (0, c)),
            pl.BlockSpec((1, 128), lambda i, c: (0, c)),
            pl.BlockSpec((128, DEC), lambda i, c: (c, 0)),
            pl.BlockSpec((D_IN, DEC), lambda i, c: (0, 0)),
            pl.BlockSpec((1, DEC), lambda i, c: (0, 0)),
        ],
        out_specs=[
            pl.BlockSpec((1, BN, 128), lambda i, c: (c, i, 0)),
            pl.BlockSpec((BN, DEC), lambda i, c: (i, 0)),
            pl.BlockSpec((BN, DEC), lambda i, c: (i, 0)),
        ],
        out_shape=[
            jax.ShapeDtypeStruct((ch, N, 128), jnp.float32),
            jax.ShapeDtypeStruct((N, DEC), jnp.float32),
            jax.ShapeDtypeStruct((N, DEC), jnp.float32),
        ],
    )(x, W1, b1, A1, dW1, db1)


def _dense2_body(agg_ref, d1_ref, w_ref, b_ref, a_ref, dw_ref, db_ref,
                 h_ref, psd_ref, dec_ref):
    c = pl.program_id(1)
    nch = pl.num_programs(1)
    d1 = d1_ref[...]
    o = jnp.zeros((BN, 128), jnp.float32)
    for k in range(DEC):
        o = o + d1[:, k][:, None] * agg_ref[k, 0, :, :]
    o = jnp.maximum(o, 0.0)
    hc = jnp.dot(o, w_ref[...], preferred_element_type=jnp.float32)
    zc = jnp.dot(o, dw_ref[...], preferred_element_type=jnp.float32)

    @pl.when(c == 0)
    def _():
        h_ref[0, :, :] = hc[:, :128]
        h_ref[1, :, :] = hc[:, 128:]
        dec_ref[...] = zc

    @pl.when(c > 0)
    def _():
        h_ref[0, :, :] += hc[:, :128]
        h_ref[1, :, :] += hc[:, 128:]
        dec_ref[...] += zc

    @pl.when(c == nch - 1)
    def _():
        b = b_ref[...]
        h0 = h_ref[0, :, :] + b[:, :128]
        h1 = h_ref[1, :, :] + b[:, 128:]
        h_ref[0, :, :] = h0
        h_ref[1, :, :] = h1
        psd_ref[...] = jnp.dot(jnp.concatenate([h0, h1], axis=1), a_ref[...],
                               preferred_element_type=jnp.float32)
        z = dec_ref[...] + db_ref[...]
        m = jnp.max(z, axis=-1, keepdims=True)
        ez = jnp.exp(z - m)
        dec_ref[...] = ez / jnp.sum(ez, axis=-1, keepdims=True)


def _dense2(agg1, dec1, W2, b2, A2, dW2, db2):
    ch_in = HID // 128
    out_ch = (HID // 2) // 128
    return pl.pallas_call(
        _dense2_body,
        grid=(NBLK, ch_in),
        in_specs=[
            pl.BlockSpec((DEC, 1, BN, 128), lambda i, c: (0, c, i, 0)),
            pl.BlockSpec((BN, DEC), lambda i, c: (i, 0)),
            pl.BlockSpec((128, HID // 2), lambda i, c: (c, 0)),
            pl.BlockSpec((1, HID // 2), lambda i, c: (0, 0)),
            pl.BlockSpec((HID // 2, DEC), lambda i, c: (0, 0)),
            pl.BlockSpec((128, DEC), lambda i, c: (c, 0)),
            pl.BlockSpec((1, DEC), lambda i, c: (0, 0)),
        ],
        out_specs=[
            pl.BlockSpec((out_ch, BN, 128), lambda i, c: (0, i, 0)),
            pl.BlockSpec((BN, DEC), lambda i, c: (i, 0)),
            pl.BlockSpec((BN, DEC), lambda i, c: (i, 0)),
        ],
        out_shape=[
            jax.ShapeDtypeStruct((out_ch, N, 128), jnp.float32),
            jax.ShapeDtypeStruct((N, DEC), jnp.float32),
            jax.ShapeDtypeStruct((N, DEC), jnp.float32),
        ],
    )(agg1, dec1, W2, b2, A2, dW2, db2)


def _final_body(agg_ref, d2_ref, fw1_ref, fb1_ref, fw2_ref, fb2_ref,
                out_ref):
    d = d2_ref[...]
    parts = []
    for c in range(2):
        o = jnp.zeros((BN, 128), jnp.float32)
        for k in range(DEC):
            o = o + d[:, k][:, None] * agg_ref[k, c, :, :]
        parts.append(o)
    o = jnp.maximum(jnp.concatenate(parts, axis=1), 0.0)
    t = jnp.maximum(
        jnp.dot(o, fw1_ref[...], preferred_element_type=jnp.float32) + fb1_ref[...],
        0.0)
    out_ref[...] = (jnp.dot(t, fw2_ref[...], preferred_element_type=jnp.float32)
                    + fb2_ref[...])


def _final(agg2, dec2, fcW1, fcb1, fcW2, fcb2):
    return pl.pallas_call(
        _final_body,
        grid=(NBLK,),
        in_specs=[
            pl.BlockSpec((DEC, 2, BN, 128), lambda i: (0, 0, i, 0)),
            pl.BlockSpec((BN, DEC), lambda i: (i, 0)),
            pl.BlockSpec((HID // 2, HID // 4), lambda i: (0, 0)),
            pl.BlockSpec((1, HID // 4), lambda i: (0, 0)),
            pl.BlockSpec((HID // 4, OUT), lambda i: (0, 0)),
            pl.BlockSpec((1, OUT), lambda i: (0, 0)),
        ],
        out_specs=pl.BlockSpec((BN, OUT), lambda i: (i, 0)),
        out_shape=jax.ShapeDtypeStruct((N, OUT), jnp.float32),
    )(agg2, dec2, fcW1, fcb1, fcW2, fcb2)


def _sc_attn_body(psd_hbm, src_hbm, dst_hbm, w_hbm,
                  psd_v, src_v, dst_v, e_v, w_v, sp_v, st_v, am_v,
                  ri_v, ro_v, ri2_v, ro2_v, maxes_sh, sparts_sh, sfull_sh):
    cid = lax.axis_index("c")
    sid = lax.axis_index("s")
    pltpu.sync_copy(psd_hbm, psd_v)
    pltpu.sync_copy(src_hbm.at[cid, sid], src_v)
    pltpu.sync_copy(dst_hbm.at[cid, sid], dst_v)

    def body1(i, mx):
        sl = pl.ds(i * 16, 16)
        sv = src_v[sl]
        dv = dst_v[sl]
        ps = plsc.load_gather(psd_v, [sv * 2])
        pd = plsc.load_gather(psd_v, [dv * 2 + 1])
        e0 = ps + pd
        e = jnp.where(e0 >= 0, e0, 0.2 * e0)
        e_v[sl] = e
        return jnp.maximum(mx, e)

    mx = lax.fori_loop(0, NV, body1, jnp.full((16,), -3e38, jnp.float32))
    st_v[...] = jnp.full((16,), jnp.max(mx, axis=0), jnp.float32)
    pltpu.sync_copy(st_v, maxes_sh.at[sid])
    plsc.subcore_barrier()
    pltpu.sync_copy(maxes_sh, am_v)

    def bodym(j, mm):
        return jnp.maximum(mm, am_v[j, :])

    mm = lax.fori_loop(0, NTILES, bodym, jnp.full((16,), -3e38, jnp.float32))
    cmax = jnp.max(mm, axis=0)

    def bodyz(i, _):
        sp_v[pl.ds(i * 16, 16)] = jnp.zeros((16,), jnp.float32)
        return 0

    lax.fori_loop(0, NV, bodyz, 0)

    def body2(i, _):
        sl = pl.ds(i * 16, 16)
        w = jnp.exp(e_v[sl] - cmax)
        w_v[sl] = w
        plsc.addupdate_scatter(sp_v, [dst_v[sl]], w)
        return 0

    lax.fori_loop(0, NV, body2, 0)
    pltpu.sync_copy(sp_v, sparts_sh.at[sid])
    plsc.subcore_barrier()

    off = sid * STR
    pltpu.sync_copy(sparts_sh.at[:, pl.ds(off, STR)], ri_v)

    def bodyr(j, _):
        sl = pl.ds(j * 16, 16)
        acc = jnp.zeros((16,), jnp.float32)
        for t in range(NTILES):
            acc = acc + ri_v[t, sl]
        ro_v[sl] = acc
        return 0

    lax.fori_loop(0, STR // 16, bodyr, 0)
    pltpu.sync_copy(ro_v, sfull_sh.at[pl.ds(off, STR)])

    @pl.when(sid == NTILES - 1)
    def _():
        pltpu.sync_copy(sparts_sh.at[:, pl.ds(NTILES * STR, 16)], ri2_v)
        acc = jnp.zeros((16,), jnp.float32)
        for t in range(NTILES):
            acc = acc + ri2_v[t, :]
        ro2_v[...] = acc
        pltpu.sync_copy(ro2_v, sfull_sh.at[pl.ds(NTILES * STR, 16)])

    plsc.subcore_barrier()
    pltpu.sync_copy(sfull_sh, sp_v)

    def body3(i, _):
        sl = pl.ds(i * 16, 16)
        sg = plsc.load_gather(sp_v, [dst_v[sl]])
        w_v[sl] = w_v[sl] / (sg + 1e-16)
        return 0

    lax.fori_loop(0, NV, body3, 0)
    pltpu.sync_copy(w_v, w_hbm.at[cid, sid])


def _sc_attn(psd, src3, dst3):
    f = functools.partial(
        pl.kernel,
        out_type=jax.ShapeDtypeStruct((DEC, NTILES, EPT), jnp.float32),
        mesh=plsc.VectorSubcoreMesh(**_SC_MESH),
        compiler_params=pltpu.CompilerParams(use_tc_tiling_on_sc=False, needs_layout_passes=False),
        scratch_types=[
            pltpu.VMEM((2 * N,), jnp.float32),
            pltpu.VMEM((EPT,), jnp.int32),
            pltpu.VMEM((EPT,), jnp.int32),
            pltpu.VMEM((EPT,), jnp.float32),
            pltpu.VMEM((EPT,), jnp.float32),
            pltpu.VMEM((EPT,), jnp.float32),
            pltpu.VMEM((16,), jnp.float32),
            pltpu.VMEM((NTILES, 16), jnp.float32),
            pltpu.VMEM((NTILES, STR), jnp.float32),
            pltpu.VMEM((STR,), jnp.float32),
            pltpu.VMEM((NTILES, 16), jnp.float32),
            pltpu.VMEM((16,), jnp.float32),
            pltpu.VMEM_SHARED((NTILES, 16), jnp.float32),
            pltpu.VMEM_SHARED((NTILES, N), jnp.float32),
            pltpu.VMEM_SHARED((N,), jnp.float32),
        ])(_sc_attn_body)
    return f(psd, src3, dst3)


def _sc_spmm_body(cpc, h_hbm, w_hbm, src_hbm, dst_hbm, agg_hbm,
                  src_v, dst_v, w_v, b0_v, b1_v, b2_v,
                  sg0, sg1, sg2, ss0, ss1, ss2, agg_sh):
    cid = lax.axis_index("c")
    sid = lax.axis_index("s")
    bufs = (b0_v, b1_v, b2_v)
    gsems = (sg0, sg1, sg2)
    ssems = (ss0, ss1, ss2)

    def zrow(r, _):
        for v in range(8):
            b0_v[r, pl.ds(v * 16, 16)] = jnp.zeros((16,), jnp.float32)
        return 0

    def scale(buf, b):
        def bodyg(g, _):
            for u in range(10):
                r = g * 10 + u
                wr = plsc.load_gather(
                    w_v, [jnp.full((16,), b * B_E + r, jnp.int32)])
                for v in range(8):
                    sl = pl.ds(v * 16, 16)
                    buf[r, sl] = buf[r, sl] * wr
            return 0

        lax.fori_loop(0, B_E // 10, bodyg, 0)

    def one_pass(p, _):
        kk = p // cpc
        ch = cid * cpc + (p % cpc)
        lax.fori_loop(0, B_E, zrow, 0)
        for q in range(6):
            pltpu.sync_copy(b0_v, agg_sh.at[pl.ds(sid * 625 + q * B_E, B_E)])
        pltpu.sync_copy(b0_v.at[pl.ds(0, 25)],
                        agg_sh.at[pl.ds(sid * 625 + 600, 25)])
        plsc.subcore_barrier()

        def quarter(q, _):
            pltpu.sync_copy(src_hbm.at[kk, sid, q], src_v)
            pltpu.sync_copy(dst_hbm.at[kk, sid, q], dst_v)
            pltpu.sync_copy(w_hbm.at[kk, sid, q], w_v)
            pltpu.async_copy(h_hbm.at[ch].at[src_v.at[0]], bufs[0], gsems[0])
            pltpu.async_copy(h_hbm.at[ch].at[src_v.at[1]], bufs[1], gsems[1])
            for b in range(NBQ):
                m = b % NBUF
                pltpu.make_async_copy(
                    h_hbm.at[ch].at[src_v.at[0]], bufs[m], gsems[m]).wait()
                scale(bufs[m], b)
                pltpu.async_copy(bufs[m], agg_sh.at[dst_v.at[b]], ssems[m],
                                 add=True)
                nb = b + 2
                if nb < NBQ:
                    mn = nb % NBUF
                    if b >= 1:
                        pltpu.make_async_copy(
                            bufs[mn], agg_sh.at[dst_v.at[0]], ssems[mn]).wait()
                    pltpu.async_copy(
                        h_hbm.at[ch].at[src_v.at[nb]], bufs[mn], gsems[mn])
            for b in range(NBQ - NBUF, NBQ):
                m = b % NBUF
                pltpu.make_async_copy(
                    bufs[m], agg_sh.at[dst_v.at[0]], ssems[m]).wait()
            return 0

        lax.fori_loop(0, NQ, quarter, 0)
        plsc.subcore_barrier()
        pltpu.sync_copy(agg_sh.at[pl.ds(sid * 625, 625)],
                        agg_hbm.at[kk, ch, pl.ds(sid * 625, 625)])
        return 0

    lax.fori_loop(0, DEC * cpc, one_pass, 0)


def _sc_spmm(h_chunks, w6, src6, dst6):
    ch = h_chunks.shape[0]
    cpc = ch // 2
    f = functools.partial(
        pl.kernel,
        out_type=jax.ShapeDtypeStruct((DEC, ch, N, 128), jnp.float32),
        mesh=plsc.VectorSubcoreMesh(**_SC_MESH),
        compiler_params=pltpu.CompilerParams(use_tc_tiling_on_sc=False, needs_layout_passes=False),
        scratch_types=[
            pltpu.VMEM((NBQ, B_E), jnp.int32),
            pltpu.VMEM((NBQ, B_E), jnp.int32),
            pltpu.VMEM((EPQ,), jnp.float32),
            pltpu.VMEM((B_E, 128), jnp.float32),
            pltpu.VMEM((B_E, 128), jnp.float32),
            pltpu.VMEM((B_E, 128), jnp.float32),
            pltpu.SemaphoreType.DMA,
            pltpu.SemaphoreType.DMA,
            pltpu.SemaphoreType.DMA,
            pltpu.SemaphoreType.DMA,
            pltpu.SemaphoreType.DMA,
            pltpu.SemaphoreType.DMA,
            pltpu.VMEM_SHARED((N, 128), jnp.float32),
        ])(functools.partial(_sc_spmm_body, cpc))
    return f(h_chunks, w6, src6, dst6)


def kernel(x, edge_index_dict, W1, b1, a_src1, a_dst1, dW1, db1,
           W2, b2, a_src2, a_dst2, dW2, db2, fcW1, fcb1, fcW2, fcb2):
    src = edge_index_dict[:, 0, :]
    dst = edge_index_dict[:, 1, :]
    src3 = src.reshape(DEC, NTILES, EPT)
    dst3 = dst.reshape(DEC, NTILES, EPT)
    src6 = src.reshape(DEC, NTILES, NQ, NBQ, B_E)
    dst6 = dst.reshape(DEC, NTILES, NQ, NBQ, B_E)

    A1 = jnp.stack([a_src1, a_dst1], axis=1)
    A2 = jnp.stack([a_src2, a_dst2], axis=1)
    b1r = b1.reshape(1, HID)
    db1r = db1.reshape(1, DEC)
    b2r = b2.reshape(1, HID // 2)
    db2r = db2.reshape(1, DEC)
    fcb1r = fcb1.reshape(1, HID // 4)
    fcb2r = fcb2.reshape(1, OUT)

    h1c, psd1, dec1 = _dense1(x, W1, b1r, A1, dW1, db1r)
    w1 = _sc_attn(psd1.reshape(2 * N), src3, dst3)
    agg1 = _sc_spmm(h1c, w1.reshape(DEC, NTILES, NQ, EPQ), src6, dst6)
    h2c, psd2, dec2 = _dense2(agg1, dec1, W2, b2r, A2, dW2, db2r)
    w2 = _sc_attn(psd2.reshape(2 * N), src3, dst3)
    agg2 = _sc_spmm(h2c, w2.reshape(DEC, NTILES, NQ, EPQ), src6, dst6)
    out = _final(agg2, dec2, fcW1, fcb1r, fcW2, fcb2r)
    return (out, dec1, dec2)


# speedup vs baseline: 2.1555x; 2.1555x over previous
import functools

import jax
import jax.numpy as jnp
from jax import lax
from jax.experimental import pallas as pl
from jax.experimental.pallas import tpu as pltpu
from jax.experimental.pallas import tpu_sc as plsc

N = 10000
E = 160000
D_IN = 256
HID = 512
OUT = 40
DEC = 2

NTILES = 16
EPT = E // NTILES
NV = EPT // 16
STR = 624
B_E = 100
NQ = 4
EPQ = EPT // NQ
NBQ = EPQ // B_E
NBUF = 3

BN = 1000
NBLK = N // BN

_SC_MESH = dict(core_axis_name="c", subcore_axis_name="s",
                num_cores=2, num_subcores=NTILES)


def _dense1_body(x_ref, w_ref, b_ref, a_ref, dw_ref, db_ref,
                 h_ref, psd_ref, dec_ref):
    c = pl.program_id(1)
    nch = pl.num_programs(1)
    xb = x_ref[...]
    hc = jnp.dot(xb, w_ref[...], preferred_element_type=jnp.float32) + b_ref[...]
    h_ref[0, :, :] = hc
    pc = jnp.dot(hc, a_ref[...], preferred_element_type=jnp.float32)

    @pl.when(c == 0)
    def _():
        psd_ref[...] = pc
        dec_ref[...] = (jnp.dot(xb, dw_ref[...], preferred_element_type=jnp.float32)
                        + db_ref[...])

    @pl.when(c > 0)
    def _():
        psd_ref[...] += pc

    @pl.when(c == nch - 1)
    def _():
        z = dec_ref[...]
        m = jnp.max(z, axis=-1, keepdims=True)
        ez = jnp.exp(z - m)
        dec_ref[...] = ez / jnp.sum(ez, axis=-1, keepdims=True)


def _dense1(x, W1, b1, A1, dW1, db1):
    ch = HID // 128
    return pl.pallas_call(
        _dense1_body,
        grid=(NBLK, ch),
        in_specs=[
            pl.BlockSpec((BN, D_IN), lambda i, c: (i, 0)),
            pl.BlockSpec((D_IN, 128), lambda i, c: (0, c)),
            pl.BlockSpec((1, 128), lambda i, c: (0, c)),
            pl.BlockSpec((128, DEC), lambda i, c: (c, 0)),
            pl.BlockSpec((D_IN, DEC), lambda i, c: (0, 0)),
            pl.BlockSpec((1, DEC), lambda i, c: (0, 0)),
        ],
        out_specs=[
            pl.BlockSpec((1, BN, 128), lambda i, c: (c, i, 0)),
            pl.BlockSpec((BN, DEC), lambda i, c: (i, 0)),
            pl.BlockSpec((BN, DEC), lambda i, c: (i, 0)),
        ],
        out_shape=[
            jax.ShapeDtypeStruct((ch, N, 128), jnp.float32),
            jax.ShapeDtypeStruct((N, DEC), jnp.float32),
            jax.ShapeDtypeStruct((N, DEC), jnp.float32),
        ],
    )(x, W1, b1, A1, dW1, db1)


def _dense2_body(agg_ref, d1_ref, w_ref, b_ref, a_ref, dw_ref, db_ref,
                 h_ref, psd_ref, dec_ref):
    c = pl.program_id(1)
    nch = pl.num_programs(1)
    d1 = d1_ref[...]
    o = jnp.zeros((BN, 128), jnp.float32)
    for k in range(DEC):
        o = o + d1[:, k][:, None] * agg_ref[k, 0, :, :]
    o = jnp.maximum(o, 0.0)
    hc = jnp.dot(o, w_ref[...], preferred_element_type=jnp.float32)
    zc = jnp.dot(o, dw_ref[...], preferred_element_type=jnp.float32)

    @pl.when(c == 0)
    def _():
        h_ref[0, :, :] = hc[:, :128]
        h_ref[1, :, :] = hc[:, 128:]
        dec_ref[...] = zc

    @pl.when(c > 0)
    def _():
        h_ref[0, :, :] += hc[:, :128]
        h_ref[1, :, :] += hc[:, 128:]
        dec_ref[...] += zc

    @pl.when(c == nch - 1)
    def _():
        b = b_ref[...]
        h0 = h_ref[0, :, :] + b[:, :128]
        h1 = h_ref[1, :, :] + b[:, 128:]
        h_ref[0, :, :] = h0
        h_ref[1, :, :] = h1
        psd_ref[...] = jnp.dot(jnp.concatenate([h0, h1], axis=1), a_ref[...],
                               preferred_element_type=jnp.float32)
        z = dec_ref[...] + db_ref[...]
        m = jnp.max(z, axis=-1, keepdims=True)
        ez = jnp.exp(z - m)
        dec_ref[...] = ez / jnp.sum(ez, axis=-1, keepdims=True)


def _dense2(agg1, dec1, W2, b2, A2, dW2, db2):
    ch_in = HID // 128
    out_ch = (HID // 2) // 128
    return pl.pallas_call(
        _dense2_body,
        grid=(NBLK, ch_in),
        in_specs=[
            pl.BlockSpec((DEC, 1, BN, 128), lambda i, c: (0, c, i, 0)),
            pl.BlockSpec((BN, DEC), lambda i, c: (i, 0)),
            pl.BlockSpec((128, HID // 2), lambda i, c: (c, 0)),
            pl.BlockSpec((1, HID // 2), lambda i, c: (0, 0)),
            pl.BlockSpec((HID // 2, DEC), lambda i, c: (0, 0)),
            pl.BlockSpec((128, DEC), lambda i, c: (c, 0)),
            pl.BlockSpec((1, DEC), lambda i, c: (0, 0)),
        ],
        out_specs=[
            pl.BlockSpec((out_ch, BN, 128), lambda i, c: (0, i, 0)),
            pl.BlockSpec((BN, DEC), lambda i, c: (i, 0)),
            pl.BlockSpec((BN, DEC), lambda i, c: (i, 0)),
        ],
        out_shape=[
            jax.ShapeDtypeStruct((out_ch, N, 128), jnp.float32),
            jax.ShapeDtypeStruct((N, DEC), jnp.float32),
            jax.ShapeDtypeStruct((N, DEC), jnp.float32),
        ],
    )(agg1, dec1, W2, b2, A2, dW2, db2)


def _final_body(agg_ref, d2_ref, fw1_ref, fb1_ref, fw2_ref, fb2_ref,
                out_ref):
    d = d2_ref[...]
    parts = []
    for c in range(2):
        o = jnp.zeros((BN, 128), jnp.float32)
        for k in range(DEC):
            o = o + d[:, k][:, None] * agg_ref[k, c, :, :]
        parts.append(o)
    o = jnp.maximum(jnp.concatenate(parts, axis=1), 0.0)
    t = jnp.maximum(
        jnp.dot(o, fw1_ref[...], preferred_element_type=jnp.float32) + fb1_ref[...],
        0.0)
    out_ref[...] = (jnp.dot(t, fw2_ref[...], preferred_element_type=jnp.float32)
                    + fb2_ref[...])


def _final(agg2, dec2, fcW1, fcb1, fcW2, fcb2):
    return pl.pallas_call(
        _final_body,
        grid=(NBLK,),
        in_specs=[
            pl.BlockSpec((DEC, 2, BN, 128), lambda i: (0, 0, i, 0)),
            pl.BlockSpec((BN, DEC), lambda i: (i, 0)),
            pl.BlockSpec((HID // 2, HID // 4), lambda i: (0, 0)),
            pl.BlockSpec((1, HID // 4), lambda i: (0, 0)),
            pl.BlockSpec((HID // 4, OUT), lambda i: (0, 0)),
            pl.BlockSpec((1, OUT), lambda i: (0, 0)),
        ],
        out_specs=pl.BlockSpec((BN, OUT), lambda i: (i, 0)),
        out_shape=jax.ShapeDtypeStruct((N, OUT), jnp.float32),
    )(agg2, dec2, fcW1, fcb1, fcW2, fcb2)


def _sc_attn_body(psd_hbm, src_hbm, dst_hbm, w_hbm,
                  psd_v, src_v, dst_v, e_v, w_v, sp_v, st_v, am_v,
                  ri_v, ro_v, ri2_v, ro2_v, maxes_sh, sparts_sh, sfull_sh):
    cid = lax.axis_index("c")
    sid = lax.axis_index("s")
    pltpu.sync_copy(psd_hbm, psd_v)
    pltpu.sync_copy(src_hbm.at[cid, sid], src_v)
    pltpu.sync_copy(dst_hbm.at[cid, sid], dst_v)

    def body1(i, mx):
        sl = pl.ds(i * 16, 16)
        sv = src_v[sl]
        dv = dst_v[sl]
        ps = plsc.load_gather(psd_v, [sv * 2])
        pd = plsc.load_gather(psd_v, [dv * 2 + 1])
        e0 = ps + pd
        e = jnp.where(e0 >= 0, e0, 0.2 * e0)
        e_v[sl] = e
        return jnp.maximum(mx, e)

    mx = lax.fori_loop(0, NV, body1, jnp.full((16,), -3e38, jnp.float32))
    st_v[...] = jnp.full((16,), jnp.max(mx, axis=0), jnp.float32)
    pltpu.sync_copy(st_v, maxes_sh.at[sid])
    plsc.subcore_barrier()
    pltpu.sync_copy(maxes_sh, am_v)

    def bodym(j, mm):
        return jnp.maximum(mm, am_v[j, :])

    mm = lax.fori_loop(0, NTILES, bodym, jnp.full((16,), -3e38, jnp.float32))
    cmax = jnp.max(mm, axis=0)

    def bodyz(i, _):
        sp_v[pl.ds(i * 16, 16)] = jnp.zeros((16,), jnp.float32)
        return 0

    lax.fori_loop(0, NV, bodyz, 0)

    def body2(i, _):
        sl = pl.ds(i * 16, 16)
        w = jnp.exp(e_v[sl] - cmax)
        w_v[sl] = w
        plsc.addupdate_scatter(sp_v, [dst_v[sl]], w)
        return 0

    lax.fori_loop(0, NV, body2, 0)
    pltpu.sync_copy(sp_v, sparts_sh.at[sid])
    plsc.subcore_barrier()

    off = sid * STR
    pltpu.sync_copy(sparts_sh.at[:, pl.ds(off, STR)], ri_v)

    def bodyr(j, _):
        sl = pl.ds(j * 16, 16)
        acc = jnp.zeros((16,), jnp.float32)
        for t in range(NTILES):
            acc = acc + ri_v[t, sl]
        ro_v[sl] = acc
        return 0

    lax.fori_loop(0, STR // 16, bodyr, 0)
    pltpu.sync_copy(ro_v, sfull_sh.at[pl.ds(off, STR)])

    @pl.when(sid == NTILES - 1)
    def _():
        pltpu.sync_copy(sparts_sh.at[:, pl.ds(NTILES * STR, 16)], ri2_v)
        acc = jnp.zeros((16,), jnp.float32)
        for t in range(NTILES):
            acc = acc + ri2_v[t, :]
        ro2_v[...] = acc
        pltpu.sync_copy(ro2_v, sfull_sh.at[pl.ds(NTILES * STR, 16)])

    plsc.subcore_barrier()
    pltpu.sync_copy(sfull_sh, sp_v)

    def body3(i, _):
        sl = pl.ds(i * 16, 16)
        sg = plsc.load_gather(sp_v, [dst_v[sl]])
        w_v[sl] = w_v[sl] / (sg + 1e-16)
        return 0

    lax.fori_loop(0, NV, body3, 0)
    pltpu.sync_copy(w_v, w_hbm.at[cid, sid])


def _sc_attn(psd, src3, dst3):
    f = functools.partial(
        pl.kernel,
        out_type=jax.ShapeDtypeStruct((DEC, NTILES, EPT), jnp.float32),
        mesh=plsc.VectorSubcoreMesh(**_SC_MESH),
        compiler_params=pltpu.CompilerParams(use_tc_tiling_on_sc=False, needs_layout_passes=False),
        scratch_types=[
            pltpu.VMEM((2 * N,), jnp.float32),
            pltpu.VMEM((EPT,), jnp.int32),
            pltpu.VMEM((EPT,), jnp.int32),
            pltpu.VMEM((EPT,), jnp.float32),
            pltpu.VMEM((EPT,), jnp.float32),
            pltpu.VMEM((EPT,), jnp.float32),
            pltpu.VMEM((16,), jnp.float32),
            pltpu.VMEM((NTILES, 16), jnp.float32),
            pltpu.VMEM((NTILES, STR), jnp.float32),
            pltpu.VMEM((STR,), jnp.float32),
            pltpu.VMEM((NTILES, 16), jnp.float32),
            pltpu.VMEM((16,), jnp.float32),
            pltpu.VMEM_SHARED((NTILES, 16), jnp.float32),
            pltpu.VMEM_SHARED((NTILES, N), jnp.float32),
            pltpu.VMEM_SHARED((N,), jnp.float32),
        ])(_sc_attn_body)
    return f(psd, src3, dst3)


def _sc_spmm_body(cpc, h_hbm, w_hbm, src_hbm, dst_hbm, agg_hbm,
                  src_v, dst_v, w_v, b0_v, b1_v, b2_v,
                  sg0, sg1, sg2, ss0, ss1, ss2, agg_sh):
    cid = lax.axis_index("c")
    sid = lax.axis_index("s")
    bufs = (b0_v, b1_v, b2_v)
    gsems = (sg0, sg1, sg2)
    ssems = (ss0, ss1, ss2)

    def zrow(r, _):
        for v in range(8):
            b0_v[r, pl.ds(v * 16, 16)] = jnp.zeros((16,), jnp.float32)
        return 0

    def scale(buf, b):
        def bodyg(g, _):
            for u in range(5):
                r = g * 5 + u
                wr = plsc.load_gather(
                    w_v, [jnp.full((16,), b * B_E + r, jnp.int32)])
                for v in range(8):
                    sl = pl.ds(v * 16, 16)
                    buf[r, sl] = buf[r, sl] * wr
            return 0

        lax.fori_loop(0, B_E // 5, bodyg, 0)

    def one_pass(p, _):
        kk = p // cpc
        ch = cid * cpc + (p % cpc)
        lax.fori_loop(0, B_E, zrow, 0)
        for q in range(6):
            pltpu.sync_copy(b0_v, agg_sh.at[pl.ds(sid * 625 + q * B_E, B_E)])
        pltpu.sync_copy(b0_v.at[pl.ds(0, 25)],
                        agg_sh.at[pl.ds(sid * 625 + 600, 25)])
        plsc.subcore_barrier()

        def quarter(q, _):
            pltpu.sync_copy(src_hbm.at[kk, sid, q], src_v)
            pltpu.sync_copy(dst_hbm.at[kk, sid, q], dst_v)
            pltpu.sync_copy(w_hbm.at[kk, sid, q], w_v)
            pltpu.async_copy(h_hbm.at[ch].at[src_v.at[0]], bufs[0], gsems[0])
            pltpu.async_copy(h_hbm.at[ch].at[src_v.at[1]], bufs[1], gsems[1])
            for b in range(NBQ):
                m = b % NBUF
                pltpu.make_async_copy(
                    h_hbm.at[ch].at[src_v.at[0]], bufs[m], gsems[m]).wait()
                scale(bufs[m], b)
                pltpu.async_copy(bufs[m], agg_sh.at[dst_v.at[b]], ssems[m],
                                 add=True)
                nb = b + 2
                if nb < NBQ:
                    mn = nb % NBUF
                    if b >= 1:
                        pltpu.make_async_copy(
                            bufs[mn], agg_sh.at[dst_v.at[0]], ssems[mn]).wait()
                    pltpu.async_copy(
                        h_hbm.at[ch].at[src_v.at[nb]], bufs[mn], gsems[mn])
            for b in range(NBQ - NBUF, NBQ):
                m = b % NBUF
                pltpu.make_async_copy(
                    bufs[m], agg_sh.at[dst_v.at[0]], ssems[m]).wait()
            return 0

        lax.fori_loop(0, NQ, quarter, 0)
        plsc.subcore_barrier()
        pltpu.sync_copy(agg_sh.at[pl.ds(sid * 625, 625)],
                        agg_hbm.at[kk, ch, pl.ds(sid * 625, 625)])
        return 0

    lax.fori_loop(0, DEC * cpc, one_pass, 0)


def _sc_spmm(h_chunks, w6, src6, dst6):
    ch = h_chunks.shape[0]
    cpc = ch // 2
    f = functools.partial(
        pl.kernel,
        out_type=jax.ShapeDtypeStruct((DEC, ch, N, 128), jnp.float32),
        mesh=plsc.VectorSubcoreMesh(**_SC_MESH),
        compiler_params=pltpu.CompilerParams(use_tc_tiling_on_sc=False, needs_layout_passes=False),
        scratch_types=[
            pltpu.VMEM((NBQ, B_E), jnp.int32),
            pltpu.VMEM((NBQ, B_E), jnp.int32),
            pltpu.VMEM((EPQ,), jnp.float32),
            pltpu.VMEM((B_E, 128), jnp.float32),
            pltpu.VMEM((B_E, 128), jnp.float32),
            pltpu.VMEM((B_E, 128), jnp.float32),
            pltpu.SemaphoreType.DMA,
            pltpu.SemaphoreType.DMA,
            pltpu.SemaphoreType.DMA,
            pltpu.SemaphoreType.DMA,
            pltpu.SemaphoreType.DMA,
            pltpu.SemaphoreType.DMA,
            pltpu.VMEM_SHARED((N, 128), jnp.float32),
        ])(functools.partial(_sc_spmm_body, cpc))
    return f(h_chunks, w6, src6, dst6)


def kernel(x, edge_index_dict, W1, b1, a_src1, a_dst1, dW1, db1,
           W2, b2, a_src2, a_dst2, dW2, db2, fcW1, fcb1, fcW2, fcb2):
    src = edge_index_dict[:, 0, :]
    dst = edge_index_dict[:, 1, :]
    src3 = src.reshape(DEC, NTILES, EPT)
    dst3 = dst.reshape(DEC, NTILES, EPT)
    src6 = src.reshape(DEC, NTILES, NQ, NBQ, B_E)
    dst6 = dst.reshape(DEC, NTILES, NQ, NBQ, B_E)

    A1 = jnp.stack([a_src1, a_dst1], axis=1)
    A2 = jnp.stack([a_src2, a_dst2], axis=1)
    b1r = b1.reshape(1, HID)
    db1r = db1.reshape(1, DEC)
    b2r = b2.reshape(1, HID // 2)
    db2r = db2.reshape(1, DEC)
    fcb1r = fcb1.reshape(1, HID // 4)
    fcb2r = fcb2.reshape(1, OUT)

    h1c, psd1, dec1 = _dense1(x, W1, b1r, A1, dW1, db1r)
    w1 = _sc_attn(psd1.reshape(2 * N), src3, dst3)
    agg1 = _sc_spmm(h1c, w1.reshape(DEC, NTILES, NQ, EPQ), src6, dst6)
    h2c, psd2, dec2 = _dense2(agg1, dec1, W2, b2r, A2, dW2, db2r)
    w2 = _sc_attn(psd2.reshape(2 * N), src3, dst3)
    agg2 = _sc_spmm(h2c, w2.reshape(DEC, NTILES, NQ, EPQ), src6, dst6)
    out = _final(agg2, dec2, fcW1, fcb1r, fcW2, fcb2r)
    return (out, dec1, dec2)


# speedup vs baseline: 2.3447x; 1.0877x over previous
import functools

import jax
import jax.numpy as jnp
from jax import lax
from jax.experimental import pallas as pl
from jax.experimental.pallas import tpu as pltpu
from jax.experimental.pallas import tpu_sc as plsc

N = 10000
E = 160000
D_IN = 256
HID = 512
OUT = 40
DEC = 2

NTILES = 16
EPT = E // NTILES
NV = EPT // 16
STR = 624
B_E = 100
NQ = 4
EPQ = EPT // NQ
NBQ = EPQ // B_E
NBUF = 3

BN = 1000
NBLK = N // BN

_SC_MESH = dict(core_axis_name="c", subcore_axis_name="s",
                num_cores=2, num_subcores=NTILES)


def _dense1_body(x_ref, w_ref, b_ref, a_ref, dw_ref, db_ref,
                 h_ref, psd_ref, dec_ref):
    c = pl.program_id(1)
    nch = pl.num_programs(1)
    xb = x_ref[...]
    hc = jnp.dot(xb, w_ref[...], preferred_element_type=jnp.float32) + b_ref[...]
    h_ref[0, :, :] = hc
    pc = jnp.dot(hc, a_ref[...], preferred_element_type=jnp.float32)

    @pl.when(c == 0)
    def _():
        psd_ref[...] = pc
        dec_ref[...] = (jnp.dot(xb, dw_ref[...], preferred_element_type=jnp.float32)
                        + db_ref[...])

    @pl.when(c > 0)
    def _():
        psd_ref[...] += pc

    @pl.when(c == nch - 1)
    def _():
        z = dec_ref[...]
        m = jnp.max(z, axis=-1, keepdims=True)
        ez = jnp.exp(z - m)
        dec_ref[...] = ez / jnp.sum(ez, axis=-1, keepdims=True)


def _dense1(x, W1, b1, A1, dW1, db1):
    ch = HID // 128
    return pl.pallas_call(
        _dense1_body,
        grid=(NBLK, ch),
        in_specs=[
            pl.BlockSpec((BN, D_IN), lambda i, c: (i, 0)),
            pl.BlockSpec((D_IN, 128), lambda i, c: (0, c)),
            pl.BlockSpec((1, 128), lambda i, c: (0, c)),
            pl.BlockSpec((128, DEC), lambda i, c: (c, 0)),
            pl.BlockSpec((D_IN, DEC), lambda i, c: (0, 0)),
            pl.BlockSpec((1, DEC), lambda i, c: (0, 0)),
        ],
        out_specs=[
            pl.BlockSpec((1, BN, 128), lambda i, c: (c, i, 0)),
            pl.BlockSpec((BN, DEC), lambda i, c: (i, 0)),
            pl.BlockSpec((BN, DEC), lambda i, c: (i, 0)),
        ],
        out_shape=[
            jax.ShapeDtypeStruct((ch, N, 128), jnp.float32),
            jax.ShapeDtypeStruct((N, DEC), jnp.float32),
            jax.ShapeDtypeStruct((N, DEC), jnp.float32),
        ],
    )(x, W1, b1, A1, dW1, db1)


def _dense2_body(agg_ref, d1_ref, w_ref, b_ref, a_ref, dw_ref, db_ref,
                 h_ref, psd_ref, dec_ref):
    c = pl.program_id(1)
    nch = pl.num_programs(1)
    d1 = d1_ref[...]
    o = jnp.zeros((BN, 128), jnp.float32)
    for k in range(DEC):
        o = o + d1[:, k][:, None] * agg_ref[k, 0, :, :]
    o = jnp.maximum(o, 0.0)
    hc = jnp.dot(o, w_ref[...], preferred_element_type=jnp.float32)
    zc = jnp.dot(o, dw_ref[...], preferred_element_type=jnp.float32)

    @pl.when(c == 0)
    def _():
        h_ref[0, :, :] = hc[:, :128]
        h_ref[1, :, :] = hc[:, 128:]
        dec_ref[...] = zc

    @pl.when(c > 0)
    def _():
        h_ref[0, :, :] += hc[:, :128]
        h_ref[1, :, :] += hc[:, 128:]
        dec_ref[...] += zc

    @pl.when(c == nch - 1)
    def _():
        b = b_ref[...]
        h0 = h_ref[0, :, :] + b[:, :128]
        h1 = h_ref[1, :, :] + b[:, 128:]
        h_ref[0, :, :] = h0
        h_ref[1, :, :] = h1
        psd_ref[...] = jnp.dot(jnp.concatenate([h0, h1], axis=1), a_ref[...],
                               preferred_element_type=jnp.float32)
        z = dec_ref[...] + db_ref[...]
        m = jnp.max(z, axis=-1, keepdims=True)
        ez = jnp.exp(z - m)
        dec_ref[...] = ez / jnp.sum(ez, axis=-1, keepdims=True)


def _dense2(agg1, dec1, W2, b2, A2, dW2, db2):
    ch_in = HID // 128
    out_ch = (HID // 2) // 128
    return pl.pallas_call(
        _dense2_body,
        grid=(NBLK, ch_in),
        in_specs=[
            pl.BlockSpec((DEC, 1, BN, 128), lambda i, c: (0, c, i, 0)),
            pl.BlockSpec((BN, DEC), lambda i, c: (i, 0)),
            pl.BlockSpec((128, HID // 2), lambda i, c: (c, 0)),
            pl.BlockSpec((1, HID // 2), lambda i, c: (0, 0)),
            pl.BlockSpec((HID // 2, DEC), lambda i, c: (0, 0)),
            pl.BlockSpec((128, DEC), lambda i, c: (c, 0)),
            pl.BlockSpec((1, DEC), lambda i, c: (0, 0)),
        ],
        out_specs=[
            pl.BlockSpec((out_ch, BN, 128), lambda i, c: (0, i, 0)),
            pl.BlockSpec((BN, DEC), lambda i, c: (i, 0)),
            pl.BlockSpec((BN, DEC), lambda i, c: (i, 0)),
        ],
        out_shape=[
            jax.ShapeDtypeStruct((out_ch, N, 128), jnp.float32),
            jax.ShapeDtypeStruct((N, DEC), jnp.float32),
            jax.ShapeDtypeStruct((N, DEC), jnp.float32),
        ],
    )(agg1, dec1, W2, b2, A2, dW2, db2)


def _final_body(agg_ref, d2_ref, fw1_ref, fb1_ref, fw2_ref, fb2_ref,
                out_ref):
    d = d2_ref[...]
    parts = []
    for c in range(2):
        o = jnp.zeros((BN, 128), jnp.float32)
        for k in range(DEC):
            o = o + d[:, k][:, None] * agg_ref[k, c, :, :]
        parts.append(o)
    o = jnp.maximum(jnp.concatenate(parts, axis=1), 0.0)
    t = jnp.maximum(
        jnp.dot(o, fw1_ref[...], preferred_element_type=jnp.float32) + fb1_ref[...],
        0.0)
    out_ref[...] = (jnp.dot(t, fw2_ref[...], preferred_element_type=jnp.float32)
                    + fb2_ref[...])


def _final(agg2, dec2, fcW1, fcb1, fcW2, fcb2):
    return pl.pallas_call(
        _final_body,
        grid=(NBLK,),
        in_specs=[
            pl.BlockSpec((DEC, 2, BN, 128), lambda i: (0, 0, i, 0)),
            pl.BlockSpec((BN, DEC), lambda i: (i, 0)),
            pl.BlockSpec((HID // 2, HID // 4), lambda i: (0, 0)),
            pl.BlockSpec((1, HID // 4), lambda i: (0, 0)),
            pl.BlockSpec((HID // 4, OUT), lambda i: (0, 0)),
            pl.BlockSpec((1, OUT), lambda i: (0, 0)),
        ],
        out_specs=pl.BlockSpec((BN, OUT), lambda i: (i, 0)),
        out_shape=jax.ShapeDtypeStruct((N, OUT), jnp.float32),
    )(agg2, dec2, fcW1, fcb1, fcW2, fcb2)


def _sc_attn_body(psd_hbm, src_hbm, dst_hbm, w_hbm,
                  psd_v, src_v, dst_v, e_v, w_v, sp_v, st_v, am_v,
                  ri_v, ro_v, ri2_v, ro2_v, maxes_sh, sparts_sh, sfull_sh):
    cid = lax.axis_index("c")
    sid = lax.axis_index("s")
    pltpu.sync_copy(psd_hbm, psd_v)
    pltpu.sync_copy(src_hbm.at[cid, sid], src_v)
    pltpu.sync_copy(dst_hbm.at[cid, sid], dst_v)

    def body1(i, mx):
        sl = pl.ds(i * 16, 16)
        sv = src_v[sl]
        dv = dst_v[sl]
        ps = plsc.load_gather(psd_v, [sv * 2])
        pd = plsc.load_gather(psd_v, [dv * 2 + 1])
        e0 = ps + pd
        e = jnp.where(e0 >= 0, e0, 0.2 * e0)
        e_v[sl] = e
        return jnp.maximum(mx, e)

    mx = lax.fori_loop(0, NV, body1, jnp.full((16,), -3e38, jnp.float32))
    st_v[...] = jnp.full((16,), jnp.max(mx, axis=0), jnp.float32)
    pltpu.sync_copy(st_v, maxes_sh.at[sid])
    plsc.subcore_barrier()
    pltpu.sync_copy(maxes_sh, am_v)

    def bodym(j, mm):
        return jnp.maximum(mm, am_v[j, :])

    mm = lax.fori_loop(0, NTILES, bodym, jnp.full((16,), -3e38, jnp.float32))
    cmax = jnp.max(mm, axis=0)

    def bodyz(i, _):
        sp_v[pl.ds(i * 16, 16)] = jnp.zeros((16,), jnp.float32)
        return 0

    lax.fori_loop(0, NV, bodyz, 0)

    def body2(i, _):
        sl = pl.ds(i * 16, 16)
        w = jnp.exp(e_v[sl] - cmax)
        w_v[sl] = w
        plsc.addupdate_scatter(sp_v, [dst_v[sl]], w)
        return 0

    lax.fori_loop(0, NV, body2, 0)
    pltpu.sync_copy(sp_v, sparts_sh.at[sid])
    plsc.subcore_barrier()

    off = sid * STR
    pltpu.sync_copy(sparts_sh.at[:, pl.ds(off, STR)], ri_v)

    def bodyr(j, _):
        sl = pl.ds(j * 16, 16)
        acc = jnp.zeros((16,), jnp.float32)
        for t in range(NTILES):
            acc = acc + ri_v[t, sl]
        ro_v[sl] = acc
        return 0

    lax.fori_loop(0, STR // 16, bodyr, 0)
    pltpu.sync_copy(ro_v, sfull_sh.at[pl.ds(off, STR)])

    @pl.when(sid == NTILES - 1)
    def _():
        pltpu.sync_copy(sparts_sh.at[:, pl.ds(NTILES * STR, 16)], ri2_v)
        acc = jnp.zeros((16,), jnp.float32)
        for t in range(NTILES):
            acc = acc + ri2_v[t, :]
        ro2_v[...] = acc
        pltpu.sync_copy(ro2_v, sfull_sh.at[pl.ds(NTILES * STR, 16)])

    plsc.subcore_barrier()
    pltpu.sync_copy(sfull_sh, sp_v)

    def body3(i, _):
        sl = pl.ds(i * 16, 16)
        sg = plsc.load_gather(sp_v, [dst_v[sl]])
        w_v[sl] = w_v[sl] / (sg + 1e-16)
        return 0

    lax.fori_loop(0, NV, body3, 0)
    pltpu.sync_copy(w_v, w_hbm.at[cid, sid])


def _sc_attn(psd, src3, dst3):
    f = functools.partial(
        pl.kernel,
        out_type=jax.ShapeDtypeStruct((DEC, NTILES, EPT), jnp.float32),
        mesh=plsc.VectorSubcoreMesh(**_SC_MESH),
        compiler_params=pltpu.CompilerParams(use_tc_tiling_on_sc=False, needs_layout_passes=False),
        scratch_types=[
            pltpu.VMEM((2 * N,), jnp.float32),
            pltpu.VMEM((EPT,), jnp.int32),
            pltpu.VMEM((EPT,), jnp.int32),
            pltpu.VMEM((EPT,), jnp.float32),
            pltpu.VMEM((EPT,), jnp.float32),
            pltpu.VMEM((EPT,), jnp.float32),
            pltpu.VMEM((16,), jnp.float32),
            pltpu.VMEM((NTILES, 16), jnp.float32),
            pltpu.VMEM((NTILES, STR), jnp.float32),
            pltpu.VMEM((STR,), jnp.float32),
            pltpu.VMEM((NTILES, 16), jnp.float32),
            pltpu.VMEM((16,), jnp.float32),
            pltpu.VMEM_SHARED((NTILES, 16), jnp.float32),
            pltpu.VMEM_SHARED((NTILES, N), jnp.float32),
            pltpu.VMEM_SHARED((N,), jnp.float32),
        ])(_sc_attn_body)
    return f(psd, src3, dst3)


def _sc_spmm_body(cpc, h_hbm, w_hbm, src_hbm, dst_hbm, agg_hbm,
                  src_v, dst_v, w_v, b0_v, b1_v, b2_v,
                  sg0, sg1, sg2, ss0, ss1, ss2, agg_sh):
    cid = lax.axis_index("c")
    sid = lax.axis_index("s")
    bufs = (b0_v, b1_v, b2_v)
    gsems = (sg0, sg1, sg2)
    ssems = (ss0, ss1, ss2)

    def zrow(r, _):
        for v in range(8):
            b0_v[r, pl.ds(v * 16, 16)] = jnp.zeros((16,), jnp.float32)
        return 0

    def scale(buf, b):
        @plsc.parallel_loop(0, B_E, step=5)
        def _(g):
            for u in range(5):
                r = g + u
                wr = plsc.load_gather(
                    w_v, [jnp.full((16,), b * B_E + r, jnp.int32)])
                for v in range(8):
                    sl = pl.ds(v * 16, 16)
                    buf[r, sl] = buf[r, sl] * wr

    def one_pass(p, _):
        kk = p // cpc
        ch = cid * cpc + (p % cpc)
        lax.fori_loop(0, B_E, zrow, 0)
        for q in range(6):
            pltpu.sync_copy(b0_v, agg_sh.at[pl.ds(sid * 625 + q * B_E, B_E)])
        pltpu.sync_copy(b0_v.at[pl.ds(0, 25)],
                        agg_sh.at[pl.ds(sid * 625 + 600, 25)])
        plsc.subcore_barrier()

        def quarter(q, _):
            pltpu.sync_copy(src_hbm.at[kk, sid, q], src_v)
            pltpu.sync_copy(dst_hbm.at[kk, sid, q], dst_v)
            pltpu.sync_copy(w_hbm.at[kk, sid, q], w_v)
            pltpu.async_copy(h_hbm.at[ch].at[src_v.at[0]], bufs[0], gsems[0])
            pltpu.async_copy(h_hbm.at[ch].at[src_v.at[1]], bufs[1], gsems[1])
            for b in range(NBQ):
                m = b % NBUF
                pltpu.make_async_copy(
                    h_hbm.at[ch].at[src_v.at[0]], bufs[m], gsems[m]).wait()
                scale(bufs[m], b)
                pltpu.async_copy(bufs[m], agg_sh.at[dst_v.at[b]], ssems[m],
                                 add=True)
                nb = b + 2
                if nb < NBQ:
                    mn = nb % NBUF
                    if b >= 1:
                        pltpu.make_async_copy(
                            bufs[mn], agg_sh.at[dst_v.at[0]], ssems[mn]).wait()
                    pltpu.async_copy(
                        h_hbm.at[ch].at[src_v.at[nb]], bufs[mn], gsems[mn])
            for b in range(NBQ - NBUF, NBQ):
                m = b % NBUF
                pltpu.make_async_copy(
                    bufs[m], agg_sh.at[dst_v.at[0]], ssems[m]).wait()
            return 0

        lax.fori_loop(0, NQ, quarter, 0)
        plsc.subcore_barrier()
        pltpu.sync_copy(agg_sh.at[pl.ds(sid * 625, 625)],
                        agg_hbm.at[kk, ch, pl.ds(sid * 625, 625)])
        return 0

    lax.fori_loop(0, DEC * cpc, one_pass, 0)


def _sc_spmm(h_chunks, w6, src6, dst6):
    ch = h_chunks.shape[0]
    cpc = ch // 2
    f = functools.partial(
        pl.kernel,
        out_type=jax.ShapeDtypeStruct((DEC, ch, N, 128), jnp.float32),
        mesh=plsc.VectorSubcoreMesh(**_SC_MESH),
        compiler_params=pltpu.CompilerParams(use_tc_tiling_on_sc=False, needs_layout_passes=False),
        scratch_types=[
            pltpu.VMEM((NBQ, B_E), jnp.int32),
            pltpu.VMEM((NBQ, B_E), jnp.int32),
            pltpu.VMEM((EPQ,), jnp.float32),
            pltpu.VMEM((B_E, 128), jnp.float32),
            pltpu.VMEM((B_E, 128), jnp.float32),
            pltpu.VMEM((B_E, 128), jnp.float32),
            pltpu.SemaphoreType.DMA,
            pltpu.SemaphoreType.DMA,
            pltpu.SemaphoreType.DMA,
            pltpu.SemaphoreType.DMA,
            pltpu.SemaphoreType.DMA,
            pltpu.SemaphoreType.DMA,
            pltpu.VMEM_SHARED((N, 128), jnp.float32),
        ])(functools.partial(_sc_spmm_body, cpc))
    return f(h_chunks, w6, src6, dst6)


def kernel(x, edge_index_dict, W1, b1, a_src1, a_dst1, dW1, db1,
           W2, b2, a_src2, a_dst2, dW2, db2, fcW1, fcb1, fcW2, fcb2):
    src = edge_index_dict[:, 0, :]
    dst = edge_index_dict[:, 1, :]
    src3 = src.reshape(DEC, NTILES, EPT)
    dst3 = dst.reshape(DEC, NTILES, EPT)
    src6 = src.reshape(DEC, NTILES, NQ, NBQ, B_E)
    dst6 = dst.reshape(DEC, NTILES, NQ, NBQ, B_E)

    A1 = jnp.stack([a_src1, a_dst1], axis=1)
    A2 = jnp.stack([a_src2, a_dst2], axis=1)
    b1r = b1.reshape(1, HID)
    db1r = db1.reshape(1, DEC)
    b2r = b2.reshape(1, HID // 2)
    db2r = db2.reshape(1, DEC)
    fcb1r = fcb1.reshape(1, HID // 4)
    fcb2r = fcb2.reshape(1, OUT)

    h1c, psd1, dec1 = _dense1(x, W1, b1r, A1, dW1, db1r)
    w1 = _sc_attn(psd1.reshape(2 * N), src3, dst3)
    agg1 = _sc_spmm(h1c, w1.reshape(DEC, NTILES, NQ, EPQ), src6, dst6)
    h2c, psd2, dec2 = _dense2(agg1, dec1, W2, b2r, A2, dW2, db2r)
    w2 = _sc_attn(psd2.reshape(2 * N), src3, dst3)
    agg2 = _sc_spmm(h2c, w2.reshape(DEC, NTILES, NQ, EPQ), src6, dst6)
    out = _final(agg2, dec2, fcW1, fcb1r, fcW2, fcb2r)
    return (out, dec1, dec2)


# speedup vs baseline: 2.3868x; 1.0180x over previous
import functools

import jax
import jax.numpy as jnp
from jax import lax
from jax.experimental import pallas as pl
from jax.experimental.pallas import tpu as pltpu
from jax.experimental.pallas import tpu_sc as plsc

N = 10000
E = 160000
D_IN = 256
HID = 512
OUT = 40
DEC = 2

NTILES = 16
EPT = E // NTILES
NV = EPT // 16
STR = 624
B_E = 100
NQ = 4
EPQ = EPT // NQ
NBQ = EPQ // B_E
NBUF = 3

BN = 1000
NBLK = N // BN

_SC_MESH = dict(core_axis_name="c", subcore_axis_name="s",
                num_cores=2, num_subcores=NTILES)


def _dense1_body(x_ref, w_ref, b_ref, a_ref, dw_ref, db_ref,
                 h_ref, psd_ref, dec_ref):
    c = pl.program_id(1)
    nch = pl.num_programs(1)
    xb = x_ref[...]
    hc = jnp.dot(xb, w_ref[...], preferred_element_type=jnp.float32) + b_ref[...]
    h_ref[0, :, :] = hc
    pc = jnp.dot(hc, a_ref[...], preferred_element_type=jnp.float32)

    @pl.when(c == 0)
    def _():
        psd_ref[...] = pc
        dec_ref[...] = (jnp.dot(xb, dw_ref[...], preferred_element_type=jnp.float32)
                        + db_ref[...])

    @pl.when(c > 0)
    def _():
        psd_ref[...] += pc

    @pl.when(c == nch - 1)
    def _():
        z = dec_ref[...]
        m = jnp.max(z, axis=-1, keepdims=True)
        ez = jnp.exp(z - m)
        dec_ref[...] = ez / jnp.sum(ez, axis=-1, keepdims=True)


def _dense1(x, W1, b1, A1, dW1, db1):
    ch = HID // 128
    return pl.pallas_call(
        _dense1_body,
        grid=(NBLK, ch),
        in_specs=[
            pl.BlockSpec((BN, D_IN), lambda i, c: (i, 0)),
            pl.BlockSpec((D_IN, 128), lambda i, c: (0, c)),
            pl.BlockSpec((1, 128), lambda i, c: (0, c)),
            pl.BlockSpec((128, DEC), lambda i, c: (c, 0)),
            pl.BlockSpec((D_IN, DEC), lambda i, c: (0, 0)),
            pl.BlockSpec((1, DEC), lambda i, c: (0, 0)),
        ],
        out_specs=[
            pl.BlockSpec((1, BN, 128), lambda i, c: (c, i, 0)),
            pl.BlockSpec((BN, DEC), lambda i, c: (i, 0)),
            pl.BlockSpec((BN, DEC), lambda i, c: (i, 0)),
        ],
        out_shape=[
            jax.ShapeDtypeStruct((ch, N, 128), jnp.float32),
            jax.ShapeDtypeStruct((N, DEC), jnp.float32),
            jax.ShapeDtypeStruct((N, DEC), jnp.float32),
        ],
    )(x, W1, b1, A1, dW1, db1)


def _dense2_body(agg_ref, d1_ref, w_ref, b_ref, a_ref, dw_ref, db_ref,
                 h_ref, psd_ref, dec_ref):
    c = pl.program_id(1)
    nch = pl.num_programs(1)
    d1 = d1_ref[...]
    o = jnp.zeros((BN, 128), jnp.float32)
    for k in range(DEC):
        o = o + d1[:, k][:, None] * agg_ref[k, 0, :, :]
    o = jnp.maximum(o, 0.0)
    hc = jnp.dot(o, w_ref[...], preferred_element_type=jnp.float32)
    zc = jnp.dot(o, dw_ref[...], preferred_element_type=jnp.float32)

    @pl.when(c == 0)
    def _():
        h_ref[0, :, :] = hc[:, :128]
        h_ref[1, :, :] = hc[:, 128:]
        dec_ref[...] = zc

    @pl.when(c > 0)
    def _():
        h_ref[0, :, :] += hc[:, :128]
        h_ref[1, :, :] += hc[:, 128:]
        dec_ref[...] += zc

    @pl.when(c == nch - 1)
    def _():
        b = b_ref[...]
        h0 = h_ref[0, :, :] + b[:, :128]
        h1 = h_ref[1, :, :] + b[:, 128:]
        h_ref[0, :, :] = h0
        h_ref[1, :, :] = h1
        psd_ref[...] = jnp.dot(jnp.concatenate([h0, h1], axis=1), a_ref[...],
                               preferred_element_type=jnp.float32)
        z = dec_ref[...] + db_ref[...]
        m = jnp.max(z, axis=-1, keepdims=True)
        ez = jnp.exp(z - m)
        dec_ref[...] = ez / jnp.sum(ez, axis=-1, keepdims=True)


def _dense2(agg1, dec1, W2, b2, A2, dW2, db2):
    ch_in = HID // 128
    out_ch = (HID // 2) // 128
    return pl.pallas_call(
        _dense2_body,
        grid=(NBLK, ch_in),
        in_specs=[
            pl.BlockSpec((DEC, 1, BN, 128), lambda i, c: (0, c, i, 0)),
            pl.BlockSpec((BN, DEC), lambda i, c: (i, 0)),
            pl.BlockSpec((128, HID // 2), lambda i, c: (c, 0)),
            pl.BlockSpec((1, HID // 2), lambda i, c: (0, 0)),
            pl.BlockSpec((HID // 2, DEC), lambda i, c: (0, 0)),
            pl.BlockSpec((128, DEC), lambda i, c: (c, 0)),
            pl.BlockSpec((1, DEC), lambda i, c: (0, 0)),
        ],
        out_specs=[
            pl.BlockSpec((out_ch, BN, 128), lambda i, c: (0, i, 0)),
            pl.BlockSpec((BN, DEC), lambda i, c: (i, 0)),
            pl.BlockSpec((BN, DEC), lambda i, c: (i, 0)),
        ],
        out_shape=[
            jax.ShapeDtypeStruct((out_ch, N, 128), jnp.float32),
            jax.ShapeDtypeStruct((N, DEC), jnp.float32),
            jax.ShapeDtypeStruct((N, DEC), jnp.float32),
        ],
    )(agg1, dec1, W2, b2, A2, dW2, db2)


def _final_body(agg_ref, d2_ref, fw1_ref, fb1_ref, fw2_ref, fb2_ref,
                out_ref):
    d = d2_ref[...]
    parts = []
    for c in range(2):
        o = jnp.zeros((BN, 128), jnp.float32)
        for k in range(DEC):
            o = o + d[:, k][:, None] * agg_ref[k, c, :, :]
        parts.append(o)
    o = jnp.maximum(jnp.concatenate(parts, axis=1), 0.0)
    t = jnp.maximum(
        jnp.dot(o, fw1_ref[...], preferred_element_type=jnp.float32) + fb1_ref[...],
        0.0)
    out_ref[...] = (jnp.dot(t, fw2_ref[...], preferred_element_type=jnp.float32)
                    + fb2_ref[...])


def _final(agg2, dec2, fcW1, fcb1, fcW2, fcb2):
    return pl.pallas_call(
        _final_body,
        grid=(NBLK,),
        in_specs=[
            pl.BlockSpec((DEC, 2, BN, 128), lambda i: (0, 0, i, 0)),
            pl.BlockSpec((BN, DEC), lambda i: (i, 0)),
            pl.BlockSpec((HID // 2, HID // 4), lambda i: (0, 0)),
            pl.BlockSpec((1, HID // 4), lambda i: (0, 0)),
            pl.BlockSpec((HID // 4, OUT), lambda i: (0, 0)),
            pl.BlockSpec((1, OUT), lambda i: (0, 0)),
        ],
        out_specs=pl.BlockSpec((BN, OUT), lambda i: (i, 0)),
        out_shape=jax.ShapeDtypeStruct((N, OUT), jnp.float32),
    )(agg2, dec2, fcW1, fcb1, fcW2, fcb2)


def _sc_attn_body(psd_hbm, src_hbm, dst_hbm, w_hbm,
                  psd_v, src_v, dst_v, e_v, w_v, sp_v, st_v, am_v,
                  ri_v, ro_v, ri2_v, ro2_v, maxes_sh, sparts_sh, sfull_sh):
    cid = lax.axis_index("c")
    sid = lax.axis_index("s")
    pltpu.sync_copy(psd_hbm, psd_v)
    pltpu.sync_copy(src_hbm.at[cid, sid], src_v)
    pltpu.sync_copy(dst_hbm.at[cid, sid], dst_v)

    @plsc.parallel_loop(0, NV, step=1,
                        carry=jnp.full((16,), -3e38, jnp.float32))
    def mx(i, mxc):
        sl = pl.ds(i * 16, 16)
        sv = src_v[sl]
        dv = dst_v[sl]
        ps = plsc.load_gather(psd_v, [sv * 2])
        pd = plsc.load_gather(psd_v, [dv * 2 + 1])
        e0 = ps + pd
        e = jnp.where(e0 >= 0, e0, 0.2 * e0)
        e_v[sl] = e
        return jnp.maximum(mxc, e)
    st_v[...] = jnp.full((16,), jnp.max(mx, axis=0), jnp.float32)
    pltpu.sync_copy(st_v, maxes_sh.at[sid])
    plsc.subcore_barrier()
    pltpu.sync_copy(maxes_sh, am_v)

    def bodym(j, mm):
        return jnp.maximum(mm, am_v[j, :])

    mm = lax.fori_loop(0, NTILES, bodym, jnp.full((16,), -3e38, jnp.float32))
    cmax = jnp.max(mm, axis=0)

    @plsc.parallel_loop(0, NV, step=4)
    def _(i):
        for u in range(4):
            sp_v[pl.ds((i + u) * 16, 16)] = jnp.zeros((16,), jnp.float32)

    def body2(i, _):
        sl = pl.ds(i * 16, 16)
        w = jnp.exp(e_v[sl] - cmax)
        w_v[sl] = w
        plsc.addupdate_scatter(sp_v, [dst_v[sl]], w)
        return 0

    lax.fori_loop(0, NV, body2, 0)
    pltpu.sync_copy(sp_v, sparts_sh.at[sid])
    plsc.subcore_barrier()

    off = sid * STR
    pltpu.sync_copy(sparts_sh.at[:, pl.ds(off, STR)], ri_v)

    @plsc.parallel_loop(0, STR // 16, step=1)
    def _(j):
        sl = pl.ds(j * 16, 16)
        acc = jnp.zeros((16,), jnp.float32)
        for t in range(NTILES):
            acc = acc + ri_v[t, sl]
        ro_v[sl] = acc
    pltpu.sync_copy(ro_v, sfull_sh.at[pl.ds(off, STR)])

    @pl.when(sid == NTILES - 1)
    def _():
        pltpu.sync_copy(sparts_sh.at[:, pl.ds(NTILES * STR, 16)], ri2_v)
        acc = jnp.zeros((16,), jnp.float32)
        for t in range(NTILES):
            acc = acc + ri2_v[t, :]
        ro2_v[...] = acc
        pltpu.sync_copy(ro2_v, sfull_sh.at[pl.ds(NTILES * STR, 16)])

    plsc.subcore_barrier()
    pltpu.sync_copy(sfull_sh, sp_v)

    @plsc.parallel_loop(0, NV, step=1)
    def _(i):
        sl = pl.ds(i * 16, 16)
        sg = plsc.load_gather(sp_v, [dst_v[sl]])
        w_v[sl] = w_v[sl] / (sg + 1e-16)
    pltpu.sync_copy(w_v, w_hbm.at[cid, sid])


def _sc_attn(psd, src3, dst3):
    f = functools.partial(
        pl.kernel,
        out_type=jax.ShapeDtypeStruct((DEC, NTILES, EPT), jnp.float32),
        mesh=plsc.VectorSubcoreMesh(**_SC_MESH),
        compiler_params=pltpu.CompilerParams(use_tc_tiling_on_sc=False, needs_layout_passes=False),
        scratch_types=[
            pltpu.VMEM((2 * N,), jnp.float32),
            pltpu.VMEM((EPT,), jnp.int32),
            pltpu.VMEM((EPT,), jnp.int32),
            pltpu.VMEM((EPT,), jnp.float32),
            pltpu.VMEM((EPT,), jnp.float32),
            pltpu.VMEM((EPT,), jnp.float32),
            pltpu.VMEM((16,), jnp.float32),
            pltpu.VMEM((NTILES, 16), jnp.float32),
            pltpu.VMEM((NTILES, STR), jnp.float32),
            pltpu.VMEM((STR,), jnp.float32),
            pltpu.VMEM((NTILES, 16), jnp.float32),
            pltpu.VMEM((16,), jnp.float32),
            pltpu.VMEM_SHARED((NTILES, 16), jnp.float32),
            pltpu.VMEM_SHARED((NTILES, N), jnp.float32),
            pltpu.VMEM_SHARED((N,), jnp.float32),
        ])(_sc_attn_body)
    return f(psd, src3, dst3)


def _sc_spmm_body(cpc, h_hbm, w_hbm, src_hbm, dst_hbm, agg_hbm,
                  src_v, dst_v, w_v, b0_v, b1_v, b2_v,
                  sg0, sg1, sg2, ss0, ss1, ss2, agg_sh):
    cid = lax.axis_index("c")
    sid = lax.axis_index("s")
    bufs = (b0_v, b1_v, b2_v)
    gsems = (sg0, sg1, sg2)
    ssems = (ss0, ss1, ss2)

    def zrow():
        @plsc.parallel_loop(0, B_E, step=1)
        def _(r):
            for v in range(8):
                b0_v[r, pl.ds(v * 16, 16)] = jnp.zeros((16,), jnp.float32)

    def scale(buf, b):
        @plsc.parallel_loop(0, B_E, step=5)
        def _(g):
            for u in range(5):
                r = g + u
                wr = plsc.load_gather(
                    w_v, [jnp.full((16,), b * B_E + r, jnp.int32)])
                for v in range(8):
                    sl = pl.ds(v * 16, 16)
                    buf[r, sl] = buf[r, sl] * wr

    def one_pass(p, _):
        kk = p // cpc
        ch = cid * cpc + (p % cpc)
        zrow()
        for q in range(6):
            pltpu.sync_copy(b0_v, agg_sh.at[pl.ds(sid * 625 + q * B_E, B_E)])
        pltpu.sync_copy(b0_v.at[pl.ds(0, 25)],
                        agg_sh.at[pl.ds(sid * 625 + 600, 25)])
        plsc.subcore_barrier()

        def quarter(q, _):
            pltpu.sync_copy(src_hbm.at[kk, sid, q], src_v)
            pltpu.sync_copy(dst_hbm.at[kk, sid, q], dst_v)
            pltpu.sync_copy(w_hbm.at[kk, sid, q], w_v)
            pltpu.async_copy(h_hbm.at[ch].at[src_v.at[0]], bufs[0], gsems[0])
            pltpu.async_copy(h_hbm.at[ch].at[src_v.at[1]], bufs[1], gsems[1])
            for b in range(NBQ):
                m = b % NBUF
                pltpu.make_async_copy(
                    h_hbm.at[ch].at[src_v.at[0]], bufs[m], gsems[m]).wait()
                scale(bufs[m], b)
                pltpu.async_copy(bufs[m], agg_sh.at[dst_v.at[b]], ssems[m],
                                 add=True)
                nb = b + 2
                if nb < NBQ:
                    mn = nb % NBUF
                    if b >= 1:
                        pltpu.make_async_copy(
                            bufs[mn], agg_sh.at[dst_v.at[0]], ssems[mn]).wait()
                    pltpu.async_copy(
                        h_hbm.at[ch].at[src_v.at[nb]], bufs[mn], gsems[mn])
            for b in range(NBQ - NBUF, NBQ):
                m = b % NBUF
                pltpu.make_async_copy(
                    bufs[m], agg_sh.at[dst_v.at[0]], ssems[m]).wait()
            return 0

        lax.fori_loop(0, NQ, quarter, 0)
        plsc.subcore_barrier()
        pltpu.sync_copy(agg_sh.at[pl.ds(sid * 625, 625)],
                        agg_hbm.at[kk, ch, pl.ds(sid * 625, 625)])
        return 0

    lax.fori_loop(0, DEC * cpc, one_pass, 0)


def _sc_spmm(h_chunks, w6, src6, dst6):
    ch = h_chunks.shape[0]
    cpc = ch // 2
    f = functools.partial(
        pl.kernel,
        out_type=jax.ShapeDtypeStruct((DEC, ch, N, 128), jnp.float32),
        mesh=plsc.VectorSubcoreMesh(**_SC_MESH),
        compiler_params=pltpu.CompilerParams(use_tc_tiling_on_sc=False, needs_layout_passes=False),
        scratch_types=[
            pltpu.VMEM((NBQ, B_E), jnp.int32),
            pltpu.VMEM((NBQ, B_E), jnp.int32),
            pltpu.VMEM((EPQ,), jnp.float32),
            pltpu.VMEM((B_E, 128), jnp.float32),
            pltpu.VMEM((B_E, 128), jnp.float32),
            pltpu.VMEM((B_E, 128), jnp.float32),
            pltpu.SemaphoreType.DMA,
            pltpu.SemaphoreType.DMA,
            pltpu.SemaphoreType.DMA,
            pltpu.SemaphoreType.DMA,
            pltpu.SemaphoreType.DMA,
            pltpu.SemaphoreType.DMA,
            pltpu.VMEM_SHARED((N, 128), jnp.float32),
        ])(functools.partial(_sc_spmm_body, cpc))
    return f(h_chunks, w6, src6, dst6)


def kernel(x, edge_index_dict, W1, b1, a_src1, a_dst1, dW1, db1,
           W2, b2, a_src2, a_dst2, dW2, db2, fcW1, fcb1, fcW2, fcb2):
    src = edge_index_dict[:, 0, :]
    dst = edge_index_dict[:, 1, :]
    src3 = src.reshape(DEC, NTILES, EPT)
    dst3 = dst.reshape(DEC, NTILES, EPT)
    src6 = src.reshape(DEC, NTILES, NQ, NBQ, B_E)
    dst6 = dst.reshape(DEC, NTILES, NQ, NBQ, B_E)

    A1 = jnp.stack([a_src1, a_dst1], axis=1)
    A2 = jnp.stack([a_src2, a_dst2], axis=1)
    b1r = b1.reshape(1, HID)
    db1r = db1.reshape(1, DEC)
    b2r = b2.reshape(1, HID // 2)
    db2r = db2.reshape(1, DEC)
    fcb1r = fcb1.reshape(1, HID // 4)
    fcb2r = fcb2.reshape(1, OUT)

    h1c, psd1, dec1 = _dense1(x, W1, b1r, A1, dW1, db1r)
    w1 = _sc_attn(psd1.reshape(2 * N), src3, dst3)
    agg1 = _sc_spmm(h1c, w1.reshape(DEC, NTILES, NQ, EPQ), src6, dst6)
    h2c, psd2, dec2 = _dense2(agg1, dec1, W2, b2r, A2, dW2, db2r)
    w2 = _sc_attn(psd2.reshape(2 * N), src3, dst3)
    agg2 = _sc_spmm(h2c, w2.reshape(DEC, NTILES, NQ, EPQ), src6, dst6)
    out = _final(agg2, dec2, fcW1, fcb1r, fcW2, fcb2r)
    return (out, dec1, dec2)


# speedup vs baseline: 2.3898x; 1.0013x over previous
import functools

import jax
import jax.numpy as jnp
from jax import lax
from jax.experimental import pallas as pl
from jax.experimental.pallas import tpu as pltpu
from jax.experimental.pallas import tpu_sc as plsc

N = 10000
E = 160000
D_IN = 256
HID = 512
OUT = 40
DEC = 2

NTILES = 16
EPT = E // NTILES
NV = EPT // 16
STR = 624
B_E = 100
NQ = 4
EPQ = EPT // NQ
NBQ = EPQ // B_E
NBUF = 3

BN = 1000
NBLK = N // BN

_SC_MESH = dict(core_axis_name="c", subcore_axis_name="s",
                num_cores=2, num_subcores=NTILES)


def _dense1_body(x_ref, w_ref, b_ref, a_ref, dw_ref, db_ref,
                 h_ref, psd_ref, dec_ref):
    c = pl.program_id(1)
    nch = pl.num_programs(1)
    xb = x_ref[...]
    hc = jnp.dot(xb, w_ref[...], preferred_element_type=jnp.float32) + b_ref[...]
    h_ref[0, :, :] = hc
    pc = jnp.dot(hc, a_ref[...], preferred_element_type=jnp.float32)

    @pl.when(c == 0)
    def _():
        psd_ref[...] = pc
        dec_ref[...] = (jnp.dot(xb, dw_ref[...], preferred_element_type=jnp.float32)
                        + db_ref[...])

    @pl.when(c > 0)
    def _():
        psd_ref[...] += pc

    @pl.when(c == nch - 1)
    def _():
        z = dec_ref[...]
        m = jnp.max(z, axis=-1, keepdims=True)
        ez = jnp.exp(z - m)
        dec_ref[...] = ez / jnp.sum(ez, axis=-1, keepdims=True)


def _dense1(x, W1, b1, A1, dW1, db1):
    ch = HID // 128
    return pl.pallas_call(
        _dense1_body,
        grid=(NBLK, ch),
        in_specs=[
            pl.BlockSpec((BN, D_IN), lambda i, c: (i, 0)),
            pl.BlockSpec((D_IN, 128), lambda i, c: (0, c)),
            pl.BlockSpec((1, 128), lambda i, c: (0, c)),
            pl.BlockSpec((128, DEC), lambda i, c: (c, 0)),
            pl.BlockSpec((D_IN, DEC), lambda i, c: (0, 0)),
            pl.BlockSpec((1, DEC), lambda i, c: (0, 0)),
        ],
        out_specs=[
            pl.BlockSpec((1, BN, 128), lambda i, c: (c, i, 0)),
            pl.BlockSpec((BN, DEC), lambda i, c: (i, 0)),
            pl.BlockSpec((BN, DEC), lambda i, c: (i, 0)),
        ],
        out_shape=[
            jax.ShapeDtypeStruct((ch, N, 128), jnp.float32),
            jax.ShapeDtypeStruct((N, DEC), jnp.float32),
            jax.ShapeDtypeStruct((N, DEC), jnp.float32),
        ],
    )(x, W1, b1, A1, dW1, db1)


def _dense2_body(agg_ref, d1_ref, w_ref, b_ref, a_ref, dw_ref, db_ref,
                 h_ref, psd_ref, dec_ref):
    c = pl.program_id(1)
    nch = pl.num_programs(1)
    d1 = d1_ref[...]
    o = jnp.zeros((BN, 128), jnp.float32)
    for k in range(DEC):
        o = o + d1[:, k][:, None] * agg_ref[k, 0, :, :]
    o = jnp.maximum(o, 0.0)
    hc = jnp.dot(o, w_ref[...], preferred_element_type=jnp.float32)
    zc = jnp.dot(o, dw_ref[...], preferred_element_type=jnp.float32)

    @pl.when(c == 0)
    def _():
        h_ref[0, :, :] = hc[:, :128]
        h_ref[1, :, :] = hc[:, 128:]
        dec_ref[...] = zc

    @pl.when(c > 0)
    def _():
        h_ref[0, :, :] += hc[:, :128]
        h_ref[1, :, :] += hc[:, 128:]
        dec_ref[...] += zc

    @pl.when(c == nch - 1)
    def _():
        b = b_ref[...]
        h0 = h_ref[0, :, :] + b[:, :128]
        h1 = h_ref[1, :, :] + b[:, 128:]
        h_ref[0, :, :] = h0
        h_ref[1, :, :] = h1
        psd_ref[...] = jnp.dot(jnp.concatenate([h0, h1], axis=1), a_ref[...],
                               preferred_element_type=jnp.float32)
        z = dec_ref[...] + db_ref[...]
        m = jnp.max(z, axis=-1, keepdims=True)
        ez = jnp.exp(z - m)
        dec_ref[...] = ez / jnp.sum(ez, axis=-1, keepdims=True)


def _dense2(agg1, dec1, W2, b2, A2, dW2, db2):
    ch_in = HID // 128
    out_ch = (HID // 2) // 128
    return pl.pallas_call(
        _dense2_body,
        grid=(NBLK, ch_in),
        in_specs=[
            pl.BlockSpec((DEC, 1, BN, 128), lambda i, c: (0, c, i, 0)),
            pl.BlockSpec((BN, DEC), lambda i, c: (i, 0)),
            pl.BlockSpec((128, HID // 2), lambda i, c: (c, 0)),
            pl.BlockSpec((1, HID // 2), lambda i, c: (0, 0)),
            pl.BlockSpec((HID // 2, DEC), lambda i, c: (0, 0)),
            pl.BlockSpec((128, DEC), lambda i, c: (c, 0)),
            pl.BlockSpec((1, DEC), lambda i, c: (0, 0)),
        ],
        out_specs=[
            pl.BlockSpec((out_ch, BN, 128), lambda i, c: (0, i, 0)),
            pl.BlockSpec((BN, DEC), lambda i, c: (i, 0)),
            pl.BlockSpec((BN, DEC), lambda i, c: (i, 0)),
        ],
        out_shape=[
            jax.ShapeDtypeStruct((out_ch, N, 128), jnp.float32),
            jax.ShapeDtypeStruct((N, DEC), jnp.float32),
            jax.ShapeDtypeStruct((N, DEC), jnp.float32),
        ],
    )(agg1, dec1, W2, b2, A2, dW2, db2)


def _final_body(agg_ref, d2_ref, fw1_ref, fb1_ref, fw2_ref, fb2_ref,
                out_ref):
    d = d2_ref[...]
    parts = []
    for c in range(2):
        o = jnp.zeros((BN, 128), jnp.float32)
        for k in range(DEC):
            o = o + d[:, k][:, None] * agg_ref[k, c, :, :]
        parts.append(o)
    o = jnp.maximum(jnp.concatenate(parts, axis=1), 0.0)
    t = jnp.maximum(
        jnp.dot(o, fw1_ref[...], preferred_element_type=jnp.float32) + fb1_ref[...],
        0.0)
    out_ref[...] = (jnp.dot(t, fw2_ref[...], preferred_element_type=jnp.float32)
                    + fb2_ref[...])


def _final(agg2, dec2, fcW1, fcb1, fcW2, fcb2):
    return pl.pallas_call(
        _final_body,
        grid=(NBLK,),
        in_specs=[
            pl.BlockSpec((DEC, 2, BN, 128), lambda i: (0, 0, i, 0)),
            pl.BlockSpec((BN, DEC), lambda i: (i, 0)),
            pl.BlockSpec((HID // 2, HID // 4), lambda i: (0, 0)),
            pl.BlockSpec((1, HID // 4), lambda i: (0, 0)),
            pl.BlockSpec((HID // 4, OUT), lambda i: (0, 0)),
            pl.BlockSpec((1, OUT), lambda i: (0, 0)),
        ],
        out_specs=pl.BlockSpec((BN, OUT), lambda i: (i, 0)),
        out_shape=jax.ShapeDtypeStruct((N, OUT), jnp.float32),
    )(agg2, dec2, fcW1, fcb1, fcW2, fcb2)


def _sc_attn_body(psd_hbm, src_hbm, dst_hbm, w_hbm,
                  psd_v, src_v, dst_v, e_v, w_v, sp_v, st_v, am_v,
                  ri_v, ro_v, ri2_v, ro2_v, maxes_sh, sparts_sh, sfull_sh):
    cid = lax.axis_index("c")
    sid = lax.axis_index("s")
    pltpu.sync_copy(psd_hbm, psd_v)
    pltpu.sync_copy(src_hbm.at[cid, sid], src_v)
    pltpu.sync_copy(dst_hbm.at[cid, sid], dst_v)

    @plsc.parallel_loop(0, NV, step=1,
                        carry=jnp.full((16,), -3e38, jnp.float32))
    def mx(i, mxc):
        sl = pl.ds(i * 16, 16)
        sv = src_v[sl]
        dv = dst_v[sl]
        ps = plsc.load_gather(psd_v, [sv * 2])
        pd = plsc.load_gather(psd_v, [dv * 2 + 1])
        e0 = ps + pd
        e = jnp.where(e0 >= 0, e0, 0.2 * e0)
        e_v[sl] = e
        return jnp.maximum(mxc, e)
    st_v[...] = jnp.full((16,), jnp.max(mx, axis=0), jnp.float32)
    pltpu.sync_copy(st_v, maxes_sh.at[sid])
    plsc.subcore_barrier()
    pltpu.sync_copy(maxes_sh, am_v)

    def bodym(j, mm):
        return jnp.maximum(mm, am_v[j, :])

    mm = lax.fori_loop(0, NTILES, bodym, jnp.full((16,), -3e38, jnp.float32))
    cmax = jnp.max(mm, axis=0)

    @plsc.parallel_loop(0, NV, step=4)
    def _(i):
        for u in range(4):
            sp_v[pl.ds((i + u) * 16, 16)] = jnp.zeros((16,), jnp.float32)

    def body2(i, _):
        sl = pl.ds(i * 16, 16)
        w = jnp.exp(e_v[sl] - cmax)
        w_v[sl] = w
        plsc.addupdate_scatter(sp_v, [dst_v[sl]], w)
        return 0

    lax.fori_loop(0, NV, body2, 0)
    pltpu.sync_copy(sp_v, sparts_sh.at[sid])
    plsc.subcore_barrier()

    off = sid * STR
    pltpu.sync_copy(sparts_sh.at[:, pl.ds(off, STR)], ri_v)

    @plsc.parallel_loop(0, STR // 16, step=1)
    def _(j):
        sl = pl.ds(j * 16, 16)
        acc = jnp.zeros((16,), jnp.float32)
        for t in range(NTILES):
            acc = acc + ri_v[t, sl]
        ro_v[sl] = acc
    pltpu.sync_copy(ro_v, sfull_sh.at[pl.ds(off, STR)])

    @pl.when(sid == NTILES - 1)
    def _():
        pltpu.sync_copy(sparts_sh.at[:, pl.ds(NTILES * STR, 16)], ri2_v)
        acc = jnp.zeros((16,), jnp.float32)
        for t in range(NTILES):
            acc = acc + ri2_v[t, :]
        ro2_v[...] = acc
        pltpu.sync_copy(ro2_v, sfull_sh.at[pl.ds(NTILES * STR, 16)])

    plsc.subcore_barrier()
    pltpu.sync_copy(sfull_sh, sp_v)

    @plsc.parallel_loop(0, NV, step=1)
    def _(i):
        sl = pl.ds(i * 16, 16)
        sg = plsc.load_gather(sp_v, [dst_v[sl]])
        w_v[sl] = w_v[sl] / (sg + 1e-16)
    pltpu.sync_copy(w_v, w_hbm.at[cid, sid])


def _sc_attn(psd, src3, dst3):
    f = functools.partial(
        pl.kernel,
        out_type=jax.ShapeDtypeStruct((DEC, NTILES, EPT), jnp.float32),
        mesh=plsc.VectorSubcoreMesh(**_SC_MESH),
        compiler_params=pltpu.CompilerParams(use_tc_tiling_on_sc=False, needs_layout_passes=False),
        scratch_types=[
            pltpu.VMEM((2 * N,), jnp.float32),
            pltpu.VMEM((EPT,), jnp.int32),
            pltpu.VMEM((EPT,), jnp.int32),
            pltpu.VMEM((EPT,), jnp.float32),
            pltpu.VMEM((EPT,), jnp.float32),
            pltpu.VMEM((EPT,), jnp.float32),
            pltpu.VMEM((16,), jnp.float32),
            pltpu.VMEM((NTILES, 16), jnp.float32),
            pltpu.VMEM((NTILES, STR), jnp.float32),
            pltpu.VMEM((STR,), jnp.float32),
            pltpu.VMEM((NTILES, 16), jnp.float32),
            pltpu.VMEM((16,), jnp.float32),
            pltpu.VMEM_SHARED((NTILES, 16), jnp.float32),
            pltpu.VMEM_SHARED((NTILES, N), jnp.float32),
            pltpu.VMEM_SHARED((N,), jnp.float32),
        ])(_sc_attn_body)
    return f(psd, src3, dst3)


def _sc_spmm_body(cpc, h_hbm, w_hbm, src_hbm, dst_hbm, agg_hbm,
                  src_v, dst_v, w_v, b0_v, b1_v, b2_v,
                  sg0, sg1, sg2, ss0, ss1, ss2, swb, agg_sh):
    cid = lax.axis_index("c")
    sid = lax.axis_index("s")
    bufs = (b0_v, b1_v, b2_v)
    gsems = (sg0, sg1, sg2)
    ssems = (ss0, ss1, ss2)

    def zrow():
        @plsc.parallel_loop(0, B_E, step=1)
        def _(r):
            for v in range(8):
                b0_v[r, pl.ds(v * 16, 16)] = jnp.zeros((16,), jnp.float32)

    def scale(buf, b):
        @plsc.parallel_loop(0, B_E, step=5)
        def _(g):
            for u in range(5):
                r = g + u
                wr = plsc.load_gather(
                    w_v, [jnp.full((16,), b * B_E + r, jnp.int32)])
                for v in range(8):
                    sl = pl.ds(v * 16, 16)
                    buf[r, sl] = buf[r, sl] * wr

    def one_pass(p, _):
        kk = p // cpc
        ch = cid * cpc + (p % cpc)

        @pl.when(p > 0)
        def _():
            pltpu.make_async_copy(agg_sh.at[pl.ds(sid * 625, 625)],
                                  agg_hbm.at[0, 0, pl.ds(sid * 625, 625)],
                                  swb).wait()

        zrow()
        for q in range(6):
            pltpu.sync_copy(b0_v, agg_sh.at[pl.ds(sid * 625 + q * B_E, B_E)])
        pltpu.sync_copy(b0_v.at[pl.ds(0, 25)],
                        agg_sh.at[pl.ds(sid * 625 + 600, 25)])
        plsc.subcore_barrier()

        def quarter(q, _):
            pltpu.sync_copy(src_hbm.at[kk, sid, q], src_v)
            pltpu.sync_copy(dst_hbm.at[kk, sid, q], dst_v)
            pltpu.sync_copy(w_hbm.at[kk, sid, q], w_v)
            pltpu.async_copy(h_hbm.at[ch].at[src_v.at[0]], bufs[0], gsems[0])
            pltpu.async_copy(h_hbm.at[ch].at[src_v.at[1]], bufs[1], gsems[1])
            for b in range(NBQ):
                m = b % NBUF
                pltpu.make_async_copy(
                    h_hbm.at[ch].at[src_v.at[0]], bufs[m], gsems[m]).wait()
                scale(bufs[m], b)
                pltpu.async_copy(bufs[m], agg_sh.at[dst_v.at[b]], ssems[m],
                                 add=True)
                nb = b + 2
                if nb < NBQ:
                    mn = nb % NBUF
                    if b >= 1:
                        pltpu.make_async_copy(
                            bufs[mn], agg_sh.at[dst_v.at[0]], ssems[mn]).wait()
                    pltpu.async_copy(
                        h_hbm.at[ch].at[src_v.at[nb]], bufs[mn], gsems[mn])
            for b in range(NBQ - NBUF, NBQ):
                m = b % NBUF
                pltpu.make_async_copy(
                    bufs[m], agg_sh.at[dst_v.at[0]], ssems[m]).wait()
            return 0

        lax.fori_loop(0, NQ, quarter, 0)
        plsc.subcore_barrier()
        pltpu.async_copy(agg_sh.at[pl.ds(sid * 625, 625)],
                         agg_hbm.at[kk, ch, pl.ds(sid * 625, 625)], swb)
        return 0

    lax.fori_loop(0, DEC * cpc, one_pass, 0)
    pltpu.make_async_copy(agg_sh.at[pl.ds(sid * 625, 625)],
                          agg_hbm.at[0, 0, pl.ds(sid * 625, 625)], swb).wait()


def _sc_spmm(h_chunks, w6, src6, dst6):
    ch = h_chunks.shape[0]
    cpc = ch // 2
    f = functools.partial(
        pl.kernel,
        out_type=jax.ShapeDtypeStruct((DEC, ch, N, 128), jnp.float32),
        mesh=plsc.VectorSubcoreMesh(**_SC_MESH),
        compiler_params=pltpu.CompilerParams(use_tc_tiling_on_sc=False, needs_layout_passes=False),
        scratch_types=[
            pltpu.VMEM((NBQ, B_E), jnp.int32),
            pltpu.VMEM((NBQ, B_E), jnp.int32),
            pltpu.VMEM((EPQ,), jnp.float32),
            pltpu.VMEM((B_E, 128), jnp.float32),
            pltpu.VMEM((B_E, 128), jnp.float32),
            pltpu.VMEM((B_E, 128), jnp.float32),
            pltpu.SemaphoreType.DMA,
            pltpu.SemaphoreType.DMA,
            pltpu.SemaphoreType.DMA,
            pltpu.SemaphoreType.DMA,
            pltpu.SemaphoreType.DMA,
            pltpu.SemaphoreType.DMA,
            pltpu.SemaphoreType.DMA,
            pltpu.VMEM_SHARED((N, 128), jnp.float32),
        ])(functools.partial(_sc_spmm_body, cpc))
    return f(h_chunks, w6, src6, dst6)


def kernel(x, edge_index_dict, W1, b1, a_src1, a_dst1, dW1, db1,
           W2, b2, a_src2, a_dst2, dW2, db2, fcW1, fcb1, fcW2, fcb2):
    src = edge_index_dict[:, 0, :]
    dst = edge_index_dict[:, 1, :]
    src3 = src.reshape(DEC, NTILES, EPT)
    dst3 = dst.reshape(DEC, NTILES, EPT)
    src6 = src.reshape(DEC, NTILES, NQ, NBQ, B_E)
    dst6 = dst.reshape(DEC, NTILES, NQ, NBQ, B_E)

    A1 = jnp.stack([a_src1, a_dst1], axis=1)
    A2 = jnp.stack([a_src2, a_dst2], axis=1)
    b1r = b1.reshape(1, HID)
    db1r = db1.reshape(1, DEC)
    b2r = b2.reshape(1, HID // 2)
    db2r = db2.reshape(1, DEC)
    fcb1r = fcb1.reshape(1, HID // 4)
    fcb2r = fcb2.reshape(1, OUT)

    h1c, psd1, dec1 = _dense1(x, W1, b1r, A1, dW1, db1r)
    w1 = _sc_attn(psd1.reshape(2 * N), src3, dst3)
    agg1 = _sc_spmm(h1c, w1.reshape(DEC, NTILES, NQ, EPQ), src6, dst6)
    h2c, psd2, dec2 = _dense2(agg1, dec1, W2, b2r, A2, dW2, db2r)
    w2 = _sc_attn(psd2.reshape(2 * N), src3, dst3)
    agg2 = _sc_spmm(h2c, w2.reshape(DEC, NTILES, NQ, EPQ), src6, dst6)
    out = _final(agg2, dec2, fcW1, fcb1r, fcW2, fcb2r)
    return (out, dec1, dec2)


# speedup vs baseline: 2.4096x; 1.0083x over previous
import functools

import jax
import jax.numpy as jnp
from jax import lax
from jax.experimental import pallas as pl
from jax.experimental.pallas import tpu as pltpu
from jax.experimental.pallas import tpu_sc as plsc

N = 10000
E = 160000
D_IN = 256
HID = 512
OUT = 40
DEC = 2

NTILES = 16
EPT = E // NTILES
NV = EPT // 16
STR = 624
B_E = 100
NQ = 4
EPQ = EPT // NQ
NBQ = EPQ // B_E
NBUF = 3

BN = 1000
NBLK = N // BN

_SC_MESH = dict(core_axis_name="c", subcore_axis_name="s",
                num_cores=2, num_subcores=NTILES)


def _dense1_body(x_ref, w_ref, b_ref, a_ref, dw_ref, db_ref,
                 h_ref, psd_ref, dec_ref):
    c = pl.program_id(1)
    nch = pl.num_programs(1)
    xb = x_ref[...]
    hc = jnp.dot(xb, w_ref[...], preferred_element_type=jnp.float32) + b_ref[...]
    h_ref[0, :, :] = hc
    pc = jnp.dot(hc, a_ref[...], preferred_element_type=jnp.float32)

    @pl.when(c == 0)
    def _():
        psd_ref[...] = pc
        dec_ref[...] = (jnp.dot(xb, dw_ref[...], preferred_element_type=jnp.float32)
                        + db_ref[...])

    @pl.when(c > 0)
    def _():
        psd_ref[...] += pc

    @pl.when(c == nch - 1)
    def _():
        z = dec_ref[...]
        m = jnp.max(z, axis=-1, keepdims=True)
        ez = jnp.exp(z - m)
        dec_ref[...] = ez / jnp.sum(ez, axis=-1, keepdims=True)


def _dense1(x, W1, b1, A1, dW1, db1):
    ch = HID // 128
    return pl.pallas_call(
        _dense1_body,
        grid=(NBLK, ch),
        in_specs=[
            pl.BlockSpec((BN, D_IN), lambda i, c: (i, 0)),
            pl.BlockSpec((D_IN, 128), lambda i, c: (0, c)),
            pl.BlockSpec((1, 128), lambda i, c: (0, c)),
            pl.BlockSpec((128, DEC), lambda i, c: (c, 0)),
            pl.BlockSpec((D_IN, DEC), lambda i, c: (0, 0)),
            pl.BlockSpec((1, DEC), lambda i, c: (0, 0)),
        ],
        out_specs=[
            pl.BlockSpec((1, BN, 128), lambda i, c: (c, i, 0)),
            pl.BlockSpec((BN, DEC), lambda i, c: (i, 0)),
            pl.BlockSpec((BN, DEC), lambda i, c: (i, 0)),
        ],
        out_shape=[
            jax.ShapeDtypeStruct((ch, N, 128), jnp.float32),
            jax.ShapeDtypeStruct((N, DEC), jnp.float32),
            jax.ShapeDtypeStruct((N, DEC), jnp.float32),
        ],
    )(x, W1, b1, A1, dW1, db1)


def _dense2_body(agg_ref, d1_ref, w_ref, b_ref, a_ref, dw_ref, db_ref,
                 h_ref, psd_ref, dec_ref):
    c = pl.program_id(1)
    nch = pl.num_programs(1)
    d1 = d1_ref[...]
    o = jnp.zeros((BN, 128), jnp.float32)
    for k in range(DEC):
        o = o + d1[:, k][:, None] * agg_ref[k, 0, :, :]
    o = jnp.maximum(o, 0.0)
    hc = jnp.dot(o, w_ref[...], preferred_element_type=jnp.float32)
    zc = jnp.dot(o, dw_ref[...], preferred_element_type=jnp.float32)

    @pl.when(c == 0)
    def _():
        h_ref[0, :, :] = hc[:, :128]
        h_ref[1, :, :] = hc[:, 128:]
        dec_ref[...] = zc

    @pl.when(c > 0)
    def _():
        h_ref[0, :, :] += hc[:, :128]
        h_ref[1, :, :] += hc[:, 128:]
        dec_ref[...] += zc

    @pl.when(c == nch - 1)
    def _():
        b = b_ref[...]
        h0 = h_ref[0, :, :] + b[:, :128]
        h1 = h_ref[1, :, :] + b[:, 128:]
        h_ref[0, :, :] = h0
        h_ref[1, :, :] = h1
        psd_ref[...] = jnp.dot(jnp.concatenate([h0, h1], axis=1), a_ref[...],
                               preferred_element_type=jnp.float32)
        z = dec_ref[...] + db_ref[...]
        m = jnp.max(z, axis=-1, keepdims=True)
        ez = jnp.exp(z - m)
        dec_ref[...] = ez / jnp.sum(ez, axis=-1, keepdims=True)


def _dense2(agg1, dec1, W2, b2, A2, dW2, db2):
    ch_in = HID // 128
    out_ch = (HID // 2) // 128
    return pl.pallas_call(
        _dense2_body,
        grid=(NBLK, ch_in),
        in_specs=[
            pl.BlockSpec((DEC, 1, BN, 128), lambda i, c: (0, c, i, 0)),
            pl.BlockSpec((BN, DEC), lambda i, c: (i, 0)),
            pl.BlockSpec((128, HID // 2), lambda i, c: (c, 0)),
            pl.BlockSpec((1, HID // 2), lambda i, c: (0, 0)),
            pl.BlockSpec((HID // 2, DEC), lambda i, c: (0, 0)),
            pl.BlockSpec((128, DEC), lambda i, c: (c, 0)),
            pl.BlockSpec((1, DEC), lambda i, c: (0, 0)),
        ],
        out_specs=[
            pl.BlockSpec((out_ch, BN, 128), lambda i, c: (0, i, 0)),
            pl.BlockSpec((BN, DEC), lambda i, c: (i, 0)),
            pl.BlockSpec((BN, DEC), lambda i, c: (i, 0)),
        ],
        out_shape=[
            jax.ShapeDtypeStruct((out_ch, N, 128), jnp.float32),
            jax.ShapeDtypeStruct((N, DEC), jnp.float32),
            jax.ShapeDtypeStruct((N, DEC), jnp.float32),
        ],
    )(agg1, dec1, W2, b2, A2, dW2, db2)


def _final_body(agg_ref, d2_ref, fw1_ref, fb1_ref, fw2_ref, fb2_ref,
                out_ref):
    d = d2_ref[...]
    parts = []
    for c in range(2):
        o = jnp.zeros((BN, 128), jnp.float32)
        for k in range(DEC):
            o = o + d[:, k][:, None] * agg_ref[k, c, :, :]
        parts.append(o)
    o = jnp.maximum(jnp.concatenate(parts, axis=1), 0.0)
    t = jnp.maximum(
        jnp.dot(o, fw1_ref[...], preferred_element_type=jnp.float32) + fb1_ref[...],
        0.0)
    out_ref[...] = (jnp.dot(t, fw2_ref[...], preferred_element_type=jnp.float32)
                    + fb2_ref[...])


def _final(agg2, dec2, fcW1, fcb1, fcW2, fcb2):
    return pl.pallas_call(
        _final_body,
        grid=(NBLK,),
        in_specs=[
            pl.BlockSpec((DEC, 2, BN, 128), lambda i: (0, 0, i, 0)),
            pl.BlockSpec((BN, DEC), lambda i: (i, 0)),
            pl.BlockSpec((HID // 2, HID // 4), lambda i: (0, 0)),
            pl.BlockSpec((1, HID // 4), lambda i: (0, 0)),
            pl.BlockSpec((HID // 4, OUT), lambda i: (0, 0)),
            pl.BlockSpec((1, OUT), lambda i: (0, 0)),
        ],
        out_specs=pl.BlockSpec((BN, OUT), lambda i: (i, 0)),
        out_shape=jax.ShapeDtypeStruct((N, OUT), jnp.float32),
    )(agg2, dec2, fcW1, fcb1, fcW2, fcb2)


def _sc_attn_body(psd_hbm, src_hbm, dst_hbm, w_hbm,
                  psd_v, src_v, dst_v, e_v, w_v, sp_v, st_v, am_v,
                  ri_v, ro_v, ri2_v, ro2_v, maxes_sh, sparts_sh, sfull_sh):
    cid = lax.axis_index("c")
    sid = lax.axis_index("s")
    pltpu.sync_copy(psd_hbm, psd_v)
    pltpu.sync_copy(src_hbm.at[cid, sid], src_v)
    pltpu.sync_copy(dst_hbm.at[cid, sid], dst_v)

    @plsc.parallel_loop(0, NV, step=1,
                        carry=jnp.full((16,), -3e38, jnp.float32))
    def mx(i, mxc):
        sl = pl.ds(i * 16, 16)
        sv = src_v[sl]
        dv = dst_v[sl]
        ps = plsc.load_gather(psd_v, [sv * 2])
        pd = plsc.load_gather(psd_v, [dv * 2 + 1])
        e0 = ps + pd
        e = jnp.where(e0 >= 0, e0, 0.2 * e0)
        e_v[sl] = e
        return jnp.maximum(mxc, e)
    st_v[...] = jnp.full((16,), jnp.max(mx, axis=0), jnp.float32)
    pltpu.sync_copy(st_v, maxes_sh.at[sid])
    plsc.subcore_barrier()
    pltpu.sync_copy(maxes_sh, am_v)

    def bodym(j, mm):
        return jnp.maximum(mm, am_v[j, :])

    mm = lax.fori_loop(0, NTILES, bodym, jnp.full((16,), -3e38, jnp.float32))
    cmax = jnp.max(mm, axis=0)

    @plsc.parallel_loop(0, NV, step=4)
    def _(i):
        for u in range(4):
            sp_v[pl.ds((i + u) * 16, 16)] = jnp.zeros((16,), jnp.float32)

    @plsc.parallel_loop(0, NV, step=1)
    def _(i):
        sl = pl.ds(i * 16, 16)
        w = jnp.exp(e_v[sl] - cmax)
        w_v[sl] = w
        plsc.addupdate_scatter(sp_v, [dst_v[sl]], w)
    pltpu.sync_copy(sp_v, sparts_sh.at[sid])
    plsc.subcore_barrier()

    off = sid * STR
    pltpu.sync_copy(sparts_sh.at[:, pl.ds(off, STR)], ri_v)

    @plsc.parallel_loop(0, STR // 16, step=1)
    def _(j):
        sl = pl.ds(j * 16, 16)
        acc = jnp.zeros((16,), jnp.float32)
        for t in range(NTILES):
            acc = acc + ri_v[t, sl]
        ro_v[sl] = acc
    pltpu.sync_copy(ro_v, sfull_sh.at[pl.ds(off, STR)])

    @pl.when(sid == NTILES - 1)
    def _():
        pltpu.sync_copy(sparts_sh.at[:, pl.ds(NTILES * STR, 16)], ri2_v)
        acc = jnp.zeros((16,), jnp.float32)
        for t in range(NTILES):
            acc = acc + ri2_v[t, :]
        ro2_v[...] = acc
        pltpu.sync_copy(ro2_v, sfull_sh.at[pl.ds(NTILES * STR, 16)])

    plsc.subcore_barrier()
    pltpu.sync_copy(sfull_sh, sp_v)

    @plsc.parallel_loop(0, NV, step=1)
    def _(i):
        sl = pl.ds(i * 16, 16)
        sg = plsc.load_gather(sp_v, [dst_v[sl]])
        w_v[sl] = w_v[sl] / (sg + 1e-16)
    pltpu.sync_copy(w_v, w_hbm.at[cid, sid])


def _sc_attn(psd, src3, dst3):
    f = functools.partial(
        pl.kernel,
        out_type=jax.ShapeDtypeStruct((DEC, NTILES, EPT), jnp.float32),
        mesh=plsc.VectorSubcoreMesh(**_SC_MESH),
        compiler_params=pltpu.CompilerParams(use_tc_tiling_on_sc=False, needs_layout_passes=False),
        scratch_types=[
            pltpu.VMEM((2 * N,), jnp.float32),
            pltpu.VMEM((EPT,), jnp.int32),
            pltpu.VMEM((EPT,), jnp.int32),
            pltpu.VMEM((EPT,), jnp.float32),
            pltpu.VMEM((EPT,), jnp.float32),
            pltpu.VMEM((EPT,), jnp.float32),
            pltpu.VMEM((16,), jnp.float32),
            pltpu.VMEM((NTILES, 16), jnp.float32),
            pltpu.VMEM((NTILES, STR), jnp.float32),
            pltpu.VMEM((STR,), jnp.float32),
            pltpu.VMEM((NTILES, 16), jnp.float32),
            pltpu.VMEM((16,), jnp.float32),
            pltpu.VMEM_SHARED((NTILES, 16), jnp.float32),
            pltpu.VMEM_SHARED((NTILES, N), jnp.float32),
            pltpu.VMEM_SHARED((N,), jnp.float32),
        ])(_sc_attn_body)
    return f(psd, src3, dst3)


def _sc_spmm_body(cpc, h_hbm, w_hbm, src_hbm, dst_hbm, agg_hbm,
                  src_v, dst_v, w_v, b0_v, b1_v, b2_v,
                  sg0, sg1, sg2, ss0, ss1, ss2, swb, agg_sh):
    cid = lax.axis_index("c")
    sid = lax.axis_index("s")
    bufs = (b0_v, b1_v, b2_v)
    gsems = (sg0, sg1, sg2)
    ssems = (ss0, ss1, ss2)

    def zrow():
        @plsc.parallel_loop(0, B_E, step=1)
        def _(r):
            for v in range(8):
                b0_v[r, pl.ds(v * 16, 16)] = jnp.zeros((16,), jnp.float32)

    def scale(buf, b):
        @plsc.parallel_loop(0, B_E, step=5)
        def _(g):
            for u in range(5):
                r = g + u
                wr = plsc.load_gather(
                    w_v, [jnp.full((16,), b * B_E + r, jnp.int32)])
                for v in range(8):
                    sl = pl.ds(v * 16, 16)
                    buf[r, sl] = buf[r, sl] * wr

    def one_pass(p, _):
        kk = p // cpc
        ch = cid * cpc + (p % cpc)

        @pl.when(p > 0)
        def _():
            pltpu.make_async_copy(agg_sh.at[pl.ds(sid * 625, 625)],
                                  agg_hbm.at[0, 0, pl.ds(sid * 625, 625)],
                                  swb).wait()

        zrow()
        for q in range(6):
            pltpu.sync_copy(b0_v, agg_sh.at[pl.ds(sid * 625 + q * B_E, B_E)])
        pltpu.sync_copy(b0_v.at[pl.ds(0, 25)],
                        agg_sh.at[pl.ds(sid * 625 + 600, 25)])
        plsc.subcore_barrier()

        def quarter(q, _):
            pltpu.sync_copy(src_hbm.at[kk, sid, q], src_v)
            pltpu.sync_copy(dst_hbm.at[kk, sid, q], dst_v)
            pltpu.sync_copy(w_hbm.at[kk, sid, q], w_v)
            pltpu.async_copy(h_hbm.at[ch].at[src_v.at[0]], bufs[0], gsems[0])
            pltpu.async_copy(h_hbm.at[ch].at[src_v.at[1]], bufs[1], gsems[1])
            for b in range(NBQ):
                m = b % NBUF
                pltpu.make_async_copy(
                    h_hbm.at[ch].at[src_v.at[0]], bufs[m], gsems[m]).wait()
                scale(bufs[m], b)
                pltpu.async_copy(bufs[m], agg_sh.at[dst_v.at[b]], ssems[m],
                                 add=True)
                nb = b + 2
                if nb < NBQ:
                    mn = nb % NBUF
                    if b >= 1:
                        pltpu.make_async_copy(
                            bufs[mn], agg_sh.at[dst_v.at[0]], ssems[mn]).wait()
                    pltpu.async_copy(
                        h_hbm.at[ch].at[src_v.at[nb]], bufs[mn], gsems[mn])
            for b in range(NBQ - NBUF, NBQ):
                m = b % NBUF
                pltpu.make_async_copy(
                    bufs[m], agg_sh.at[dst_v.at[0]], ssems[m]).wait()
            return 0

        lax.fori_loop(0, NQ, quarter, 0)
        plsc.subcore_barrier()
        pltpu.async_copy(agg_sh.at[pl.ds(sid * 625, 625)],
                         agg_hbm.at[kk, ch, pl.ds(sid * 625, 625)], swb)
        return 0

    lax.fori_loop(0, DEC * cpc, one_pass, 0)
    pltpu.make_async_copy(agg_sh.at[pl.ds(sid * 625, 625)],
                          agg_hbm.at[0, 0, pl.ds(sid * 625, 625)], swb).wait()


def _sc_spmm(h_chunks, w6, src6, dst6):
    ch = h_chunks.shape[0]
    cpc = ch // 2
    f = functools.partial(
        pl.kernel,
        out_type=jax.ShapeDtypeStruct((DEC, ch, N, 128), jnp.float32),
        mesh=plsc.VectorSubcoreMesh(**_SC_MESH),
        compiler_params=pltpu.CompilerParams(use_tc_tiling_on_sc=False, needs_layout_passes=False),
        scratch_types=[
            pltpu.VMEM((NBQ, B_E), jnp.int32),
            pltpu.VMEM((NBQ, B_E), jnp.int32),
            pltpu.VMEM((EPQ,), jnp.float32),
            pltpu.VMEM((B_E, 128), jnp.float32),
            pltpu.VMEM((B_E, 128), jnp.float32),
            pltpu.VMEM((B_E, 128), jnp.float32),
            pltpu.SemaphoreType.DMA,
            pltpu.SemaphoreType.DMA,
            pltpu.SemaphoreType.DMA,
            pltpu.SemaphoreType.DMA,
            pltpu.SemaphoreType.DMA,
            pltpu.SemaphoreType.DMA,
            pltpu.SemaphoreType.DMA,
            pltpu.VMEM_SHARED((N, 128), jnp.float32),
        ])(functools.partial(_sc_spmm_body, cpc))
    return f(h_chunks, w6, src6, dst6)


def kernel(x, edge_index_dict, W1, b1, a_src1, a_dst1, dW1, db1,
           W2, b2, a_src2, a_dst2, dW2, db2, fcW1, fcb1, fcW2, fcb2):
    src = edge_index_dict[:, 0, :]
    dst = edge_index_dict[:, 1, :]
    src3 = src.reshape(DEC, NTILES, EPT)
    dst3 = dst.reshape(DEC, NTILES, EPT)
    src6 = src.reshape(DEC, NTILES, NQ, NBQ, B_E)
    dst6 = dst.reshape(DEC, NTILES, NQ, NBQ, B_E)

    A1 = jnp.stack([a_src1, a_dst1], axis=1)
    A2 = jnp.stack([a_src2, a_dst2], axis=1)
    b1r = b1.reshape(1, HID)
    db1r = db1.reshape(1, DEC)
    b2r = b2.reshape(1, HID // 2)
    db2r = db2.reshape(1, DEC)
    fcb1r = fcb1.reshape(1, HID // 4)
    fcb2r = fcb2.reshape(1, OUT)

    h1c, psd1, dec1 = _dense1(x, W1, b1r, A1, dW1, db1r)
    w1 = _sc_attn(psd1.reshape(2 * N), src3, dst3)
    agg1 = _sc_spmm(h1c, w1.reshape(DEC, NTILES, NQ, EPQ), src6, dst6)
    h2c, psd2, dec2 = _dense2(agg1, dec1, W2, b2r, A2, dW2, db2r)
    w2 = _sc_attn(psd2.reshape(2 * N), src3, dst3)
    agg2 = _sc_spmm(h2c, w2.reshape(DEC, NTILES, NQ, EPQ), src6, dst6)
    out = _final(agg2, dec2, fcW1, fcb1r, fcW2, fcb2r)
    return (out, dec1, dec2)
